# Initial kernel scaffold; baseline (speedup 1.0000x reference)
import jax
import jax.numpy as jnp
from jax.experimental import pallas as pl


def kernel(pred_mask, features, N):
    raise NotImplementedError("write your pallas kernel here")


# speedup vs baseline: 1.0112x; 1.0112x over previous
import jax
import jax.numpy as jnp
from jax.experimental import pallas as pl
from jax.experimental.pallas import tpu as pltpu

_NEG = -1e9


def _stage1_body(pm_ref, masked_ref):
    x = pm_ref[0, 0]
    H, W = x.shape
    unc = -jnp.abs(x)
    binm = (x > 0.0).astype(jnp.float32)

    def shift_rows(a, d, fill):
        f = jnp.full((1, a.shape[1]), fill, a.dtype)
        if d == 1:
            return jnp.concatenate([a[1:, :], f], axis=0)
        else:
            return jnp.concatenate([f, a[:-1, :]], axis=0)

    def shift_cols(a, d, fill):
        f = jnp.full((a.shape[0], 1), fill, a.dtype)
        if d == 1:
            return jnp.concatenate([a[:, 1:], f], axis=1)
        else:
            return jnp.concatenate([f, a[:, :-1]], axis=1)

    def pool3(a, op, fill):
        h = op(a, op(shift_cols(a, 1, fill), shift_cols(a, -1, fill)))
        return op(h, op(shift_rows(h, 1, fill), shift_rows(h, -1, fill)))

    dil = pool3(binm, jnp.maximum, -jnp.inf)
    ero = pool3(binm, jnp.minimum, jnp.inf)
    edge = (dil != ero).astype(jnp.float32)
    edge2 = pool3(edge, jnp.maximum, -jnp.inf) > 0.0
    masked_ref[0] = jnp.where(edge2, unc, jnp.full_like(unc, _NEG))


def _stage1(pred_mask):
    B, _, H, W = pred_mask.shape
    return pl.pallas_call(
        _stage1_body,
        grid=(B,),
        in_specs=[pl.BlockSpec((1, 1, H, W), lambda b: (b, 0, 0, 0))],
        out_specs=pl.BlockSpec((1, H, W), lambda b: (b, 0, 0)),
        out_shape=jax.ShapeDtypeStruct((B, H, W), jnp.float32),
    )(pred_mask)


def _bilinear_sample(fm, grid):
    C, H, W = fm.shape
    gx = grid[:, 0]
    gy = grid[:, 1]
    ix = ((gx + 1.0) * W - 1.0) / 2.0
    iy = ((gy + 1.0) * H - 1.0) / 2.0
    x0 = jnp.floor(ix)
    x1 = x0 + 1.0
    y0 = jnp.floor(iy)
    y1 = y0 + 1.0
    wx1 = ix - x0
    wx0 = 1.0 - wx1
    wy1 = iy - y0
    wy0 = 1.0 - wy1

    def gather(xc, yc):
        valid = (xc >= 0) & (xc <= W - 1) & (yc >= 0) & (yc <= H - 1)
        xcc = jnp.clip(xc, 0, W - 1).astype(jnp.int32)
        ycc = jnp.clip(yc, 0, H - 1).astype(jnp.int32)
        v = fm[:, ycc, xcc]
        return v * valid.astype(fm.dtype)[None, :]

    out = (gather(x0, y0) * (wx0 * wy0)[None, :]
           + gather(x1, y0) * (wx1 * wy0)[None, :]
           + gather(x0, y1) * (wx0 * wy1)[None, :]
           + gather(x1, y1) * (wx1 * wy1)[None, :])
    return out


def kernel(pred_mask, features, N):
    B, C, H, W = features.shape
    masked = _stage1(pred_mask)

    n_static = 4096
    vals, idx = jax.lax.top_k(masked.reshape(B, -1), n_static)

    ys = (idx // W).astype(jnp.float32) / float(H - 1)
    xs = (idx % W).astype(jnp.float32) / float(W - 1)
    coords = jnp.stack([ys, xs], axis=-1)

    grid = 2.0 * coords - 1.0
    sampled = jax.vmap(_bilinear_sample)(features, grid)
    sampled = jnp.transpose(sampled, (0, 2, 1))

    out = jnp.concatenate([vals[..., None], coords, sampled], axis=-1)
    return out


# speedup vs baseline: 3.4420x; 3.4037x over previous
import functools

import jax
import jax.numpy as jnp
from jax import lax
from jax.experimental import pallas as pl
from jax.experimental.pallas import tpu as pltpu
from jax.experimental.pallas import tpu_sc as plsc

_NEG = -1e9
B, C, H, W = 2, 96, 384, 384
HW = H * W
K = 4096
NT = 16
CH = HW // NT
NV = CH // 16
PPT = K // NT
TRASH = K
LOOKAHEAD = 6


def _stage1_body(pm_ref, ukey_ref):
    x = pm_ref[0, 0]
    unc = -jnp.abs(x)
    binm = (x > 0.0).astype(jnp.float32)

    def shift_rows(a, d, fill):
        f = jnp.full((1, a.shape[1]), fill, a.dtype)
        if d == 1:
            return jnp.concatenate([a[1:, :], f], axis=0)
        return jnp.concatenate([f, a[:-1, :]], axis=0)

    def shift_cols(a, d, fill):
        f = jnp.full((a.shape[0], 1), fill, a.dtype)
        if d == 1:
            return jnp.concatenate([a[:, 1:], f], axis=1)
        return jnp.concatenate([f, a[:, :-1]], axis=1)

    def pool3(a, op, fill):
        h = op(a, op(shift_cols(a, 1, fill), shift_cols(a, -1, fill)))
        return op(h, op(shift_rows(h, 1, fill), shift_rows(h, -1, fill)))

    dil = pool3(binm, jnp.maximum, -jnp.inf)
    ero = pool3(binm, jnp.minimum, jnp.inf)
    edge = (dil != ero).astype(jnp.float32)
    edge2 = pool3(edge, jnp.maximum, -jnp.inf) > 0.0
    masked = jnp.where(edge2, unc, jnp.full_like(unc, _NEG))
    ukey_ref[0] = lax.bitcast_convert_type(masked, jnp.int32) & jnp.int32(0x7FFFFFFF)


def _stage1(pred_mask):
    return pl.pallas_call(
        _stage1_body,
        grid=(B,),
        in_specs=[pl.BlockSpec((1, 1, H, W), lambda b: (b, 0, 0, 0))],
        out_specs=pl.BlockSpec((1, H, W), lambda b: (b, 0, 0)),
        out_shape=jax.ShapeDtypeStruct((B, H, W), jnp.int32),
    )(pred_mask)


def _sc_body(ukeys, feats, vals_o, ys_o, xs_o, samp_o,
             keys_v, hist_v, lhist_v, ghist_v, sel_k_v, sel_i_v, eq_i_v,
             sidx_v, tkey_v, cnts_v, tmp16_v, sk_v, si_v, pk_v, pi_v,
             ys0_v, xs0_v, ca_v, w00_v, w01_v, w10_v, w11_v,
             vals_v, ysn_v, xsn_v, gbuf_v, srow_v,
             hist_s, cnt_s, gk_s, gi_s, exk_s, exi_s,
             sem_in, sem_out):
    cid = lax.axis_index("c")
    sid = lax.axis_index("s")
    lanes = lax.iota(jnp.int32, 16)
    z16 = jnp.zeros((16,), jnp.int32)
    o16 = z16 + 1
    ones_i = o16

    pltpu.sync_copy(ukeys.at[cid, pl.ds(sid * CH, CH)], keys_v)

    prefix = jnp.int32(0)
    kk = jnp.int32(K)
    for p in range(4):
        s = 24 - 8 * p

        def zero_body(i, _):
            hist_v[pl.ds(i * 16, 16)] = z16
            return 0
        lax.fori_loop(0, 256, zero_body, 0)

        def scan_body(i, _):
            v = keys_v[pl.ds(i * 16, 16)]
            dig = (v >> s) & 255
            if p == 0:
                act = v >= 0
            else:
                act = (v >> (s + 8)) == prefix
            plsc.addupdate_scatter(hist_v, [lanes * 256 + dig], ones_i, mask=act)
            return 0
        lax.fori_loop(0, NV, scan_body, 0)

        def red_body(j, _):
            acc = z16
            for l in range(16):
                acc = acc + hist_v[pl.ds(l * 256 + j * 16, 16)]
            lhist_v[pl.ds(j * 16, 16)] = acc
            return 0
        lax.fori_loop(0, 16, red_body, 0)

        pltpu.sync_copy(lhist_v, hist_s.at[sid])
        plsc.subcore_barrier()
        pltpu.sync_copy(hist_s, ghist_v)
        plsc.subcore_barrier()

        def find_body(j, carry):
            csum, dstar, below = carry
            g = z16
            for l in range(16):
                g = g + ghist_v[l, pl.ds(j * 16, 16)]
            cum = jnp.cumsum(g) + csum
            mlt = cum < kk
            dstar = dstar + jnp.sum(mlt.astype(jnp.int32))
            below = below + jnp.sum(jnp.where(mlt, g, 0))
            csum = csum + jnp.sum(g)
            return csum, dstar, below
        _, dstar, below = lax.fori_loop(
            0, 16, find_body, (jnp.int32(0), jnp.int32(0), jnp.int32(0)))
        prefix = prefix * 256 + dstar
        kk = kk - below

    T = prefix
    need_eq = kk

    def comp_body(i, carry):
        c_lt, c_eq = carry
        v = keys_v[pl.ds(i * 16, 16)]
        idxv = sid * CH + i * 16 + lanes
        m_lt = v < T
        m_eq = v == T
        plsc.store_compressed(sel_k_v.at[pl.ds(c_lt, 16)], v, mask=m_lt)
        plsc.store_compressed(sel_i_v.at[pl.ds(c_lt, 16)], idxv, mask=m_lt)
        plsc.store_compressed(eq_i_v.at[pl.ds(c_eq, 16)], idxv, mask=m_eq)
        c_lt = c_lt + jnp.sum(m_lt.astype(jnp.int32))
        c_eq = c_eq + jnp.sum(m_eq.astype(jnp.int32))
        return c_lt, c_eq
    c_lt, c_eq = lax.fori_loop(0, NV, comp_body, (jnp.int32(0), jnp.int32(0)))

    tmp16_v[pl.ds(0, 16)] = jnp.where(lanes == 0, c_lt,
                                      jnp.where(lanes == 1, c_eq, 0))
    pltpu.sync_copy(tmp16_v, cnt_s.at[sid])
    plsc.subcore_barrier()
    pltpu.sync_copy(cnt_s, cnts_v)

    nlt_vec = plsc.load_gather(cnts_v, [lanes, z16])
    neq_vec = plsc.load_gather(cnts_v, [lanes, o16])
    before = lanes < sid
    off_lt = jnp.sum(jnp.where(before, nlt_vec, 0))
    n_lt_all = jnp.sum(nlt_vec)
    off_eq = jnp.sum(jnp.where(before, neq_vec, 0))
    m_me = jnp.clip(need_eq - off_eq, 0, c_eq)

    def fill_tk(i, _):
        tkey_v[pl.ds(i * 16, 16)] = z16 + T
        return 0
    lax.fori_loop(0, 8, fill_tk, 0)

    def pack_chunks(n_items, dst_base, src_k, src_i):
        nch = (n_items + 127) // 128

        def chunk_body(cc, _):
            rem = n_items - cc * 128
            for g in range(8):
                rvec = g * 16 + lanes
                dest = jnp.where(rvec < rem, dst_base + cc * 128 + rvec,
                                 TRASH + rvec)
                sidx_v[0, pl.ds(g * 16, 16)] = dest
            if src_k is None:
                pltpu.async_copy(tkey_v, gk_s.at[sidx_v.at[0]],
                                 sem_in.at[0]).wait()
            else:
                pltpu.async_copy(src_k.at[pl.ds(cc * 128, 128)],
                                 gk_s.at[sidx_v.at[0]], sem_in.at[0]).wait()
            pltpu.async_copy(src_i.at[pl.ds(cc * 128, 128)],
                             gi_s.at[sidx_v.at[0]], sem_in.at[0]).wait()
            return 0
        lax.fori_loop(0, nch, chunk_body, 0)

    pack_chunks(c_lt, off_lt, sel_k_v, sel_i_v)
    pack_chunks(m_me, n_lt_all + off_eq, None, eq_i_v)
    plsc.subcore_barrier()

    pltpu.sync_copy(gk_s.at[pl.ds(sid * PPT, PPT)], sk_v)
    pltpu.sync_copy(gi_s.at[pl.ds(sid * PPT, PPT)], si_v)

    def keep(sel, a, b):
        return jnp.where(sel, a, b)

    for k2 in [2 << a for a in range(12)]:
        j = k2 >> 1
        while j >= 1:
            if j >= PPT:
                jb = j // PPT
                pltpu.sync_copy(sk_v, exk_s.at[sid])
                pltpu.sync_copy(si_v, exi_s.at[sid])
                plsc.subcore_barrier()
                partner = jnp.bitwise_xor(sid, jb)
                pltpu.sync_copy(exk_s.at[partner], pk_v)
                pltpu.sync_copy(exi_s.at[partner], pi_v)
                plsc.subcore_barrier()
                i_low = jnp.bitwise_and(sid, jb) == 0
                up = jnp.bitwise_and(sid * PPT, k2) == 0
                want_min = up == i_low

                def x_body(g, _):
                    a_k = sk_v[pl.ds(g * 16, 16)]
                    a_i = si_v[pl.ds(g * 16, 16)]
                    b_k = pk_v[pl.ds(g * 16, 16)]
                    b_i = pi_v[pl.ds(g * 16, 16)]
                    agtb = (a_k > b_k) | ((a_k == b_k) & (a_i > b_i))
                    sel = agtb != want_min
                    sk_v[pl.ds(g * 16, 16)] = keep(sel, a_k, b_k)
                    si_v[pl.ds(g * 16, 16)] = keep(sel, a_i, b_i)
                    return 0
                lax.fori_loop(0, 16, x_body, 0)
            elif j >= 16:
                jv = j // 16

                def m_body(pp, _):
                    v = jnp.bitwise_or(jnp.bitwise_and(pp, jv - 1),
                                       (pp & ~(jv - 1)) << 1)
                    a_k = sk_v[pl.ds(v * 16, 16)]
                    a_i = si_v[pl.ds(v * 16, 16)]
                    b_k = sk_v[pl.ds((v + jv) * 16, 16)]
                    b_i = si_v[pl.ds((v + jv) * 16, 16)]
                    agtb = (a_k > b_k) | ((a_k == b_k) & (a_i > b_i))
                    up = jnp.bitwise_and(sid * PPT + v * 16, k2) == 0
                    sel_lo = agtb != up
                    sk_v[pl.ds(v * 16, 16)] = keep(sel_lo, a_k, b_k)
                    si_v[pl.ds(v * 16, 16)] = keep(sel_lo, a_i, b_i)
                    sk_v[pl.ds((v + jv) * 16, 16)] = keep(sel_lo, b_k, a_k)
                    si_v[pl.ds((v + jv) * 16, 16)] = keep(sel_lo, b_i, a_i)
                    return 0
                lax.fori_loop(0, 8, m_body, 0)
            else:
                lxj = jnp.bitwise_xor(lanes, j)
                i_low_v = jnp.bitwise_and(lanes, j) == 0

                def v_body(g, _):
                    xk = sk_v[pl.ds(g * 16, 16)]
                    xi = si_v[pl.ds(g * 16, 16)]
                    pk = plsc.load_gather(sk_v, [g * 16 + lxj])
                    pi = plsc.load_gather(si_v, [g * 16 + lxj])
                    agtb = (xk > pk) | ((xk == pk) & (xi > pi))
                    if k2 >= 16:
                        upv = jnp.full((16,), False) | (
                            jnp.bitwise_and(sid * PPT + g * 16, k2) == 0)
                    else:
                        upv = jnp.bitwise_and(lanes, k2) == 0
                    want_min_v = upv == i_low_v
                    sel = agtb != want_min_v
                    sk_v[pl.ds(g * 16, 16)] = keep(sel, xk, pk)
                    si_v[pl.ds(g * 16, 16)] = keep(sel, xi, pi)
                    return 0
                lax.fori_loop(0, 16, v_body, 0)
            j >>= 1

    def parm_body(g, _):
        ky = sk_v[pl.ds(g * 16, 16)]
        idx = si_v[pl.ds(g * 16, 16)]
        valf = plsc.bitcast(ky | jnp.int32(-2147483648), jnp.float32)
        y = ((idx >> 7) * 21846) >> 16
        x = idx - y * 384
        ysn = y.astype(jnp.float32) / jnp.float32(H - 1)
        xsn = x.astype(jnp.float32) / jnp.float32(W - 1)
        gx = 2.0 * ysn - 1.0
        gy = 2.0 * xsn - 1.0
        iy = ((gy + 1.0) * H - 1.0) / 2.0
        ix = ((gx + 1.0) * W - 1.0) / 2.0
        y0 = (iy + 1.0).astype(jnp.int32) - 1
        x0 = (ix + 1.0).astype(jnp.int32) - 1
        wy1 = iy - y0.astype(jnp.float32)
        wx1 = ix - x0.astype(jnp.float32)
        wy0 = 1.0 - wy1
        wx0 = 1.0 - wx1
        ys0 = jnp.clip(y0, 0, H - 2)
        xs0 = jnp.clip(x0, 0, W - 2)
        zf = jnp.zeros((16,), jnp.float32)
        wyA = jnp.where(y0 == ys0, wy0, zf) + jnp.where(y0 + 1 == ys0, wy1, zf)
        wyB = (jnp.where(y0 == ys0 + 1, wy0, zf)
               + jnp.where(y0 + 1 == ys0 + 1, wy1, zf))
        wxA = jnp.where(x0 == xs0, wx0, zf) + jnp.where(x0 + 1 == xs0, wx1, zf)
        wxB = (jnp.where(x0 == xs0 + 1, wx0, zf)
               + jnp.where(x0 + 1 == xs0 + 1, wx1, zf))
        xa = jnp.minimum(xs0 & ~7, W - 16)
        sl = pl.ds(g * 16, 16)
        vals_v[sl] = valf
        ysn_v[sl] = ysn
        xsn_v[sl] = xsn
        ys0_v[sl] = ys0
        xs0_v[sl] = xa
        ca_v[sl] = xs0 - xa
        w00_v[sl] = wxA * wyA
        w01_v[sl] = wxB * wyA
        w10_v[sl] = wxA * wyB
        w11_v[sl] = wxB * wyB
        return 0
    lax.fori_loop(0, 16, parm_body, 0)

    pltpu.sync_copy(vals_v, vals_o.at[cid, pl.ds(sid * PPT, PPT)])
    pltpu.sync_copy(ysn_v, ys_o.at[cid, pl.ds(sid * PPT, PPT)])
    pltpu.sync_copy(xsn_v, xs_o.at[cid, pl.ds(sid * PPT, PPT)])

    def sld(ref, i):
        return ref[pl.ds(i, 16)][0]

    def in_copy(p):
        slot = jnp.bitwise_and(p, 7)
        return pltpu.make_async_copy(
            feats.at[cid, :, pl.ds(sld(ys0_v, p), 2),
                     pl.ds(pl.multiple_of(sld(xs0_v, p), 8), 16)],
            gbuf_v.at[slot], sem_in.at[slot])

    def out_copy(p):
        slot = jnp.bitwise_and(p, 7)
        return pltpu.make_async_copy(
            srow_v.at[slot], samp_o.at[cid, sid * PPT + p], sem_out.at[slot])

    def pro_body(q, _):
        in_copy(q).start()
        return 0
    lax.fori_loop(0, LOOKAHEAD, pro_body, 0)

    def g_body(p, _):
        @pl.when(p + LOOKAHEAD < PPT)
        def _():
            in_copy(p + LOOKAHEAD).start()
        in_copy(p).wait()

        @pl.when(p >= 8)
        def _():
            out_copy(p - 8).wait()
        slot = jnp.bitwise_and(p, 7)
        rv = z16 + slot
        w00 = sld(w00_v, p)
        w01 = sld(w01_v, p)
        w10 = sld(w10_v, p)
        w11 = sld(w11_v, p)
        cav = z16 + sld(ca_v, p)
        cav1 = cav + 1
        for cb in range(6):
            cvec = cb * 16 + lanes
            v00 = plsc.load_gather(gbuf_v, [rv, cvec, z16, cav])
            v01 = plsc.load_gather(gbuf_v, [rv, cvec, z16, cav1])
            v10 = plsc.load_gather(gbuf_v, [rv, cvec, o16, cav])
            v11 = plsc.load_gather(gbuf_v, [rv, cvec, o16, cav1])
            acc = v00 * w00 + v01 * w01 + v10 * w10 + v11 * w11
            srow_v[slot, pl.ds(cb * 16, 16)] = acc
        out_copy(p).start()
        return 0
    lax.fori_loop(0, PPT, g_body, 0)

    def drain_body(q, _):
        out_copy(PPT - 8 + q).wait()
        return 0
    lax.fori_loop(0, 8, drain_body, 0)


@functools.partial(jax.jit, static_argnames=())
def _sc_topk_gather(ukeys, feats):
    mesh = plsc.VectorSubcoreMesh(core_axis_name="c", subcore_axis_name="s")
    f = pl.kernel(
        _sc_body,
        out_type=(
            jax.ShapeDtypeStruct((B, K), jnp.float32),
            jax.ShapeDtypeStruct((B, K), jnp.float32),
            jax.ShapeDtypeStruct((B, K), jnp.float32),
            jax.ShapeDtypeStruct((B, K, C), jnp.float32),
        ),
        mesh=mesh,
        compiler_params=pltpu.CompilerParams(use_tc_tiling_on_sc=False,
                                             needs_layout_passes=False),
        scratch_types=[
            pltpu.VMEM((CH,), jnp.int32),
            pltpu.VMEM((4096,), jnp.int32),
            pltpu.VMEM((256,), jnp.int32),
            pltpu.VMEM((16, 256), jnp.int32),
            pltpu.VMEM((4224,), jnp.int32),
            pltpu.VMEM((4224,), jnp.int32),
            pltpu.VMEM((CH + 144,), jnp.int32),
            pltpu.VMEM((2, 128), jnp.int32),
            pltpu.VMEM((128,), jnp.int32),
            pltpu.VMEM((16, 16), jnp.int32),
            pltpu.VMEM((16,), jnp.int32),
            pltpu.VMEM((PPT,), jnp.int32),
            pltpu.VMEM((PPT,), jnp.int32),
            pltpu.VMEM((PPT,), jnp.int32),
            pltpu.VMEM((PPT,), jnp.int32),
            pltpu.VMEM((PPT + 16,), jnp.int32),
            pltpu.VMEM((PPT + 16,), jnp.int32),
            pltpu.VMEM((PPT + 16,), jnp.int32),
            pltpu.VMEM((PPT + 16,), jnp.float32),
            pltpu.VMEM((PPT + 16,), jnp.float32),
            pltpu.VMEM((PPT + 16,), jnp.float32),
            pltpu.VMEM((PPT + 16,), jnp.float32),
            pltpu.VMEM((PPT,), jnp.float32),
            pltpu.VMEM((PPT,), jnp.float32),
            pltpu.VMEM((PPT,), jnp.float32),
            pltpu.VMEM((8, C, 2, 16), jnp.float32),
            pltpu.VMEM((8, C), jnp.float32),
            pltpu.VMEM_SHARED((16, 256), jnp.int32),
            pltpu.VMEM_SHARED((16, 16), jnp.int32),
            pltpu.VMEM_SHARED((K + 128, ), jnp.int32),
            pltpu.VMEM_SHARED((K + 128, ), jnp.int32),
            pltpu.VMEM_SHARED((16, 256), jnp.int32),
            pltpu.VMEM_SHARED((16, 256), jnp.int32),
            pltpu.SemaphoreType.DMA((8,)),
            pltpu.SemaphoreType.DMA((8,)),
        ],
    )
    return f(ukeys, feats)


def kernel(pred_mask, features, N):
    ukeys = _stage1(pred_mask).reshape(B, HW)
    vals, ysn, xsn, samp = _sc_topk_gather(ukeys, features)
    out = jnp.concatenate(
        [vals[..., None], ysn[..., None], xsn[..., None], samp], axis=-1)
    return out


# speedup vs baseline: 4.2370x; 1.2310x over previous
import jax
import jax.numpy as jnp
from jax import lax
from jax.experimental import pallas as pl
from jax.experimental.pallas import tpu as pltpu
from jax.experimental.pallas import tpu_sc as plsc

_NEG = -1e9
B, C, H, W = 2, 96, 384, 384
HW = H * W
K = 4096
NT = 16
CH = HW // NT
NV = CH // 16
PPT = K // NT
TRASH = K
LOOKAHEAD = 6


def _stage1_body(pm_ref, ukey_ref):
    x = pm_ref[0, 0]
    unc = -jnp.abs(x)
    binm = (x > 0.0).astype(jnp.float32)

    def shift_rows(a, d, fill):
        f = jnp.full((1, a.shape[1]), fill, a.dtype)
        if d == 1:
            return jnp.concatenate([a[1:, :], f], axis=0)
        return jnp.concatenate([f, a[:-1, :]], axis=0)

    def shift_cols(a, d, fill):
        f = jnp.full((a.shape[0], 1), fill, a.dtype)
        if d == 1:
            return jnp.concatenate([a[:, 1:], f], axis=1)
        return jnp.concatenate([f, a[:, :-1]], axis=1)

    def pool3(a, op, fill):
        h = op(a, op(shift_cols(a, 1, fill), shift_cols(a, -1, fill)))
        return op(h, op(shift_rows(h, 1, fill), shift_rows(h, -1, fill)))

    dil = pool3(binm, jnp.maximum, -jnp.inf)
    ero = pool3(binm, jnp.minimum, jnp.inf)
    edge = (dil != ero).astype(jnp.float32)
    edge2 = pool3(edge, jnp.maximum, -jnp.inf) > 0.0
    masked = jnp.where(edge2, unc, jnp.full_like(unc, _NEG))
    ukey_ref[0] = lax.bitcast_convert_type(masked, jnp.int32) & jnp.int32(0x7FFFFFFF)


def _stage1(pred_mask):
    return pl.pallas_call(
        _stage1_body,
        grid=(B,),
        in_specs=[pl.BlockSpec((1, 1, H, W), lambda b: (b, 0, 0, 0))],
        out_specs=pl.BlockSpec((1, H, W), lambda b: (b, 0, 0)),
        out_shape=jax.ShapeDtypeStruct((B, H, W), jnp.int32),
    )(pred_mask)


def _sel_body(ukeys, vals_o, ys_o, xs_o, sidx_o,
              keys_v, hist_v, lhist_v, ghist_v, sel_k_v, sel_i_v, eq_i_v,
              sidx_v, tkey_v, cnts_v, tmp16_v, sk_v, si_v, pk_v, pi_v,
              vals_v, ysn_v, xsn_v,
              hist_s, cnt_s, gk_s, gi_s, exk_s, exi_s, sem_d):
    cid = lax.axis_index("c")
    sid = lax.axis_index("s")
    lanes = lax.iota(jnp.int32, 16)
    z16 = jnp.zeros((16,), jnp.int32)
    o16 = z16 + 1
    ones_i = o16

    pltpu.sync_copy(ukeys.at[cid, pl.ds(sid * CH, CH)], keys_v)

    prefix = jnp.int32(0)
    kk = jnp.int32(K)
    for p in range(4):
        s = 24 - 8 * p

        def zero_body(i, _):
            hist_v[pl.ds(i * 16, 16)] = z16
            return 0
        lax.fori_loop(0, 256, zero_body, 0)

        def scan_body(i, _):
            v = keys_v[pl.ds(i * 16, 16)]
            dig = (v >> s) & 255
            if p == 0:
                act = v >= 0
            else:
                act = (v >> (s + 8)) == prefix
            plsc.addupdate_scatter(hist_v, [lanes * 256 + dig], ones_i, mask=act)
            return 0
        lax.fori_loop(0, NV, scan_body, 0)

        def red_body(j, _):
            acc = z16
            for l in range(16):
                acc = acc + hist_v[pl.ds(l * 256 + j * 16, 16)]
            lhist_v[pl.ds(j * 16, 16)] = acc
            return 0
        lax.fori_loop(0, 16, red_body, 0)

        pltpu.sync_copy(lhist_v, hist_s.at[sid])
        plsc.subcore_barrier()
        pltpu.sync_copy(hist_s, ghist_v)
        plsc.subcore_barrier()

        def find_body(j, carry):
            csum, dstar, below = carry
            g = z16
            for l in range(16):
                g = g + ghist_v[l, pl.ds(j * 16, 16)]
            cum = jnp.cumsum(g) + csum
            mlt = cum < kk
            dstar = dstar + jnp.sum(mlt.astype(jnp.int32))
            below = below + jnp.sum(jnp.where(mlt, g, 0))
            csum = csum + jnp.sum(g)
            return csum, dstar, below
        _, dstar, below = lax.fori_loop(
            0, 16, find_body, (jnp.int32(0), jnp.int32(0), jnp.int32(0)))
        prefix = prefix * 256 + dstar
        kk = kk - below

    T = prefix
    need_eq = kk

    def comp_body(i, carry):
        c_lt, c_eq = carry
        v = keys_v[pl.ds(i * 16, 16)]
        idxv = sid * CH + i * 16 + lanes
        m_lt = v < T
        m_eq = v == T
        plsc.store_compressed(sel_k_v.at[pl.ds(c_lt, 16)], v, mask=m_lt)
        plsc.store_compressed(sel_i_v.at[pl.ds(c_lt, 16)], idxv, mask=m_lt)
        plsc.store_compressed(eq_i_v.at[pl.ds(c_eq, 16)], idxv, mask=m_eq)
        c_lt = c_lt + jnp.sum(m_lt.astype(jnp.int32))
        c_eq = c_eq + jnp.sum(m_eq.astype(jnp.int32))
        return c_lt, c_eq
    c_lt, c_eq = lax.fori_loop(0, NV, comp_body, (jnp.int32(0), jnp.int32(0)))

    tmp16_v[pl.ds(0, 16)] = jnp.where(lanes == 0, c_lt,
                                      jnp.where(lanes == 1, c_eq, 0))
    pltpu.sync_copy(tmp16_v, cnt_s.at[sid])
    plsc.subcore_barrier()
    pltpu.sync_copy(cnt_s, cnts_v)

    nlt_vec = plsc.load_gather(cnts_v, [lanes, z16])
    neq_vec = plsc.load_gather(cnts_v, [lanes, o16])
    before = lanes < sid
    off_lt = jnp.sum(jnp.where(before, nlt_vec, 0))
    n_lt_all = jnp.sum(nlt_vec)
    off_eq = jnp.sum(jnp.where(before, neq_vec, 0))
    m_me = jnp.clip(need_eq - off_eq, 0, c_eq)

    def fill_tk(i, _):
        tkey_v[pl.ds(i * 16, 16)] = z16 + T
        return 0
    lax.fori_loop(0, 8, fill_tk, 0)

    def pack_chunks(n_items, dst_base, src_k, src_i):
        nch = (n_items + 127) // 128

        def chunk_body(cc, _):
            rem = n_items - cc * 128
            for g in range(8):
                rvec = g * 16 + lanes
                dest = jnp.where(rvec < rem, dst_base + cc * 128 + rvec,
                                 TRASH + rvec)
                sidx_v[0, pl.ds(g * 16, 16)] = dest
            if src_k is None:
                pltpu.async_copy(tkey_v, gk_s.at[sidx_v.at[0]],
                                 sem_d.at[0]).wait()
            else:
                pltpu.async_copy(src_k.at[pl.ds(cc * 128, 128)],
                                 gk_s.at[sidx_v.at[0]], sem_d.at[0]).wait()
            pltpu.async_copy(src_i.at[pl.ds(cc * 128, 128)],
                             gi_s.at[sidx_v.at[0]], sem_d.at[0]).wait()
            return 0
        lax.fori_loop(0, nch, chunk_body, 0)

    pack_chunks(c_lt, off_lt, sel_k_v, sel_i_v)
    pack_chunks(m_me, n_lt_all + off_eq, None, eq_i_v)
    plsc.subcore_barrier()

    pltpu.sync_copy(gk_s.at[pl.ds(sid * PPT, PPT)], sk_v)
    pltpu.sync_copy(gi_s.at[pl.ds(sid * PPT, PPT)], si_v)

    def keep(sel, a, b):
        return jnp.where(sel, a, b)

    for k2 in [2 << a for a in range(12)]:
        j = k2 >> 1
        while j >= 1:
            if j >= PPT:
                jb = j // PPT
                pltpu.sync_copy(sk_v, exk_s.at[sid])
                pltpu.sync_copy(si_v, exi_s.at[sid])
                plsc.subcore_barrier()
                partner = jnp.bitwise_xor(sid, jb)
                pltpu.sync_copy(exk_s.at[partner], pk_v)
                pltpu.sync_copy(exi_s.at[partner], pi_v)
                plsc.subcore_barrier()
                i_low = jnp.bitwise_and(sid, jb) == 0
                up = jnp.bitwise_and(sid * PPT, k2) == 0
                want_min = up == i_low

                def x_body(g, _):
                    a_k = sk_v[pl.ds(g * 16, 16)]
                    a_i = si_v[pl.ds(g * 16, 16)]
                    b_k = pk_v[pl.ds(g * 16, 16)]
                    b_i = pi_v[pl.ds(g * 16, 16)]
                    agtb = (a_k > b_k) | ((a_k == b_k) & (a_i > b_i))
                    sel = agtb != want_min
                    sk_v[pl.ds(g * 16, 16)] = keep(sel, a_k, b_k)
                    si_v[pl.ds(g * 16, 16)] = keep(sel, a_i, b_i)
                    return 0
                lax.fori_loop(0, 16, x_body, 0)
            elif j >= 16:
                jv = j // 16

                def m_body(pp, _):
                    v = jnp.bitwise_or(jnp.bitwise_and(pp, jv - 1),
                                       (pp & ~(jv - 1)) << 1)
                    a_k = sk_v[pl.ds(v * 16, 16)]
                    a_i = si_v[pl.ds(v * 16, 16)]
                    b_k = sk_v[pl.ds((v + jv) * 16, 16)]
                    b_i = si_v[pl.ds((v + jv) * 16, 16)]
                    agtb = (a_k > b_k) | ((a_k == b_k) & (a_i > b_i))
                    up = jnp.bitwise_and(sid * PPT + v * 16, k2) == 0
                    sel_lo = agtb != up
                    sk_v[pl.ds(v * 16, 16)] = keep(sel_lo, a_k, b_k)
                    si_v[pl.ds(v * 16, 16)] = keep(sel_lo, a_i, b_i)
                    sk_v[pl.ds((v + jv) * 16, 16)] = keep(sel_lo, b_k, a_k)
                    si_v[pl.ds((v + jv) * 16, 16)] = keep(sel_lo, b_i, a_i)
                    return 0
                lax.fori_loop(0, 8, m_body, 0)
            else:
                lxj = jnp.bitwise_xor(lanes, j)
                i_low_v = jnp.bitwise_and(lanes, j) == 0

                def v_body(g, _):
                    xk = sk_v[pl.ds(g * 16, 16)]
                    xi = si_v[pl.ds(g * 16, 16)]
                    pk = plsc.load_gather(sk_v, [g * 16 + lxj])
                    pi = plsc.load_gather(si_v, [g * 16 + lxj])
                    agtb = (xk > pk) | ((xk == pk) & (xi > pi))
                    if k2 >= 16:
                        upv = jnp.full((16,), False) | (
                            jnp.bitwise_and(sid * PPT + g * 16, k2) == 0)
                    else:
                        upv = jnp.bitwise_and(lanes, k2) == 0
                    want_min_v = upv == i_low_v
                    sel = agtb != want_min_v
                    sk_v[pl.ds(g * 16, 16)] = keep(sel, xk, pk)
                    si_v[pl.ds(g * 16, 16)] = keep(sel, xi, pi)
                    return 0
                lax.fori_loop(0, 16, v_body, 0)
            j >>= 1

    def out_body(g, _):
        ky = sk_v[pl.ds(g * 16, 16)]
        idx = si_v[pl.ds(g * 16, 16)]
        valf = plsc.bitcast(ky | jnp.int32(-2147483648), jnp.float32)
        y = ((idx >> 7) * 21846) >> 16
        x = idx - y * 384
        sl = pl.ds(g * 16, 16)
        vals_v[sl] = valf
        ysn_v[sl] = y.astype(jnp.float32) / jnp.float32(H - 1)
        xsn_v[sl] = x.astype(jnp.float32) / jnp.float32(W - 1)
        return 0
    lax.fori_loop(0, 16, out_body, 0)

    pltpu.sync_copy(vals_v, vals_o.at[cid, pl.ds(sid * PPT, PPT)])
    pltpu.sync_copy(ysn_v, ys_o.at[cid, pl.ds(sid * PPT, PPT)])
    pltpu.sync_copy(xsn_v, xs_o.at[cid, pl.ds(sid * PPT, PPT)])
    pltpu.sync_copy(si_v, sidx_o.at[cid, pl.ds(sid * PPT, PPT)])


def _sc_select(ukeys):
    mesh = plsc.VectorSubcoreMesh(core_axis_name="c", subcore_axis_name="s")
    f = pl.kernel(
        _sel_body,
        out_type=(
            jax.ShapeDtypeStruct((B, K), jnp.float32),
            jax.ShapeDtypeStruct((B, K), jnp.float32),
            jax.ShapeDtypeStruct((B, K), jnp.float32),
            jax.ShapeDtypeStruct((B, K), jnp.int32),
        ),
        mesh=mesh,
        compiler_params=pltpu.CompilerParams(use_tc_tiling_on_sc=False,
                                             needs_layout_passes=False),
        scratch_types=[
            pltpu.VMEM((CH,), jnp.int32),
            pltpu.VMEM((4096,), jnp.int32),
            pltpu.VMEM((256,), jnp.int32),
            pltpu.VMEM((16, 256), jnp.int32),
            pltpu.VMEM((4224,), jnp.int32),
            pltpu.VMEM((4224,), jnp.int32),
            pltpu.VMEM((CH + 144,), jnp.int32),
            pltpu.VMEM((2, 128), jnp.int32),
            pltpu.VMEM((128,), jnp.int32),
            pltpu.VMEM((16, 16), jnp.int32),
            pltpu.VMEM((16,), jnp.int32),
            pltpu.VMEM((PPT,), jnp.int32),
            pltpu.VMEM((PPT,), jnp.int32),
            pltpu.VMEM((PPT,), jnp.int32),
            pltpu.VMEM((PPT,), jnp.int32),
            pltpu.VMEM((PPT,), jnp.float32),
            pltpu.VMEM((PPT,), jnp.float32),
            pltpu.VMEM((PPT,), jnp.float32),
            pltpu.VMEM_SHARED((16, 256), jnp.int32),
            pltpu.VMEM_SHARED((16, 16), jnp.int32),
            pltpu.VMEM_SHARED((K + 128,), jnp.int32),
            pltpu.VMEM_SHARED((K + 128,), jnp.int32),
            pltpu.VMEM_SHARED((16, 256), jnp.int32),
            pltpu.VMEM_SHARED((16, 256), jnp.int32),
            pltpu.SemaphoreType.DMA((2,)),
        ],
    )
    return f(ukeys)


def _gat_body(sidx, feats, samp_o,
              si_v, ys0_v, xs0_v, ca_v, w00_v, w01_v, w10_v, w11_v,
              gbuf_v, srow_v, sem_in, sem_out):
    cid = lax.axis_index("c")
    sid = lax.axis_index("s")
    lanes = lax.iota(jnp.int32, 16)
    z16 = jnp.zeros((16,), jnp.int32)
    o16 = z16 + 1

    pltpu.sync_copy(sidx.at[cid, pl.ds(sid * PPT, PPT)], si_v)

    def parm_body(g, _):
        idx = si_v[pl.ds(g * 16, 16)]
        y = ((idx >> 7) * 21846) >> 16
        x = idx - y * 384
        ysn = y.astype(jnp.float32) / jnp.float32(H - 1)
        xsn = x.astype(jnp.float32) / jnp.float32(W - 1)
        gx = 2.0 * ysn - 1.0
        gy = 2.0 * xsn - 1.0
        iy = ((gy + 1.0) * H - 1.0) / 2.0
        ix = ((gx + 1.0) * W - 1.0) / 2.0
        y0 = (iy + 1.0).astype(jnp.int32) - 1
        x0 = (ix + 1.0).astype(jnp.int32) - 1
        wy1 = iy - y0.astype(jnp.float32)
        wx1 = ix - x0.astype(jnp.float32)
        wy0 = 1.0 - wy1
        wx0 = 1.0 - wx1
        ys0 = jnp.clip(y0, 0, H - 2)
        xs0 = jnp.clip(x0, 0, W - 2)
        zf = jnp.zeros((16,), jnp.float32)
        wyA = jnp.where(y0 == ys0, wy0, zf) + jnp.where(y0 + 1 == ys0, wy1, zf)
        wyB = (jnp.where(y0 == ys0 + 1, wy0, zf)
               + jnp.where(y0 + 1 == ys0 + 1, wy1, zf))
        wxA = jnp.where(x0 == xs0, wx0, zf) + jnp.where(x0 + 1 == xs0, wx1, zf)
        wxB = (jnp.where(x0 == xs0 + 1, wx0, zf)
               + jnp.where(x0 + 1 == xs0 + 1, wx1, zf))
        xa = jnp.where((xs0 & 15) == 15, xs0 - 7, xs0 & ~15)
        sl = pl.ds(g * 16, 16)
        ys0_v[sl] = ys0
        xs0_v[sl] = xa
        ca_v[sl] = xs0 - xa
        w00_v[sl] = wxA * wyA
        w01_v[sl] = wxB * wyA
        w10_v[sl] = wxA * wyB
        w11_v[sl] = wxB * wyB
        return 0
    lax.fori_loop(0, 16, parm_body, 0)

    def sld(ref, i):
        return ref[pl.ds(i, 16)][0]

    def in_copy(p):
        slot = jnp.bitwise_and(p, 7)
        return pltpu.make_async_copy(
            feats.at[cid, :, pl.ds(sld(ys0_v, p), 2),
                     pl.ds(pl.multiple_of(sld(xs0_v, p), 8), 16)],
            gbuf_v.at[slot], sem_in.at[slot])

    def out_copy(p):
        slot = jnp.bitwise_and(p, 7)
        return pltpu.make_async_copy(
            srow_v.at[slot], samp_o.at[cid, sid * PPT + p], sem_out.at[slot])

    def pro_body(q, _):
        in_copy(q).start()
        return 0
    lax.fori_loop(0, LOOKAHEAD, pro_body, 0)

    def g_body(p, _):
        @pl.when(p + LOOKAHEAD < PPT)
        def _():
            in_copy(p + LOOKAHEAD).start()
        in_copy(p).wait()

        @pl.when(p >= 8)
        def _():
            out_copy(p - 8).wait()
        slot = jnp.bitwise_and(p, 7)
        rv = z16 + slot
        w00 = sld(w00_v, p)
        w01 = sld(w01_v, p)
        w10 = sld(w10_v, p)
        w11 = sld(w11_v, p)
        cav = z16 + sld(ca_v, p)
        cav1 = cav + 1
        for cb in range(6):
            cvec = cb * 16 + lanes
            v00 = plsc.load_gather(gbuf_v, [rv, cvec, z16, cav])
            v01 = plsc.load_gather(gbuf_v, [rv, cvec, z16, cav1])
            v10 = plsc.load_gather(gbuf_v, [rv, cvec, o16, cav])
            v11 = plsc.load_gather(gbuf_v, [rv, cvec, o16, cav1])
            acc = v00 * w00 + v01 * w01 + v10 * w10 + v11 * w11
            srow_v[slot, pl.ds(cb * 16, 16)] = acc
        out_copy(p).start()
        return 0
    lax.fori_loop(0, PPT, g_body, 0)

    def drain_body(q, _):
        out_copy(PPT - 8 + q).wait()
        return 0
    lax.fori_loop(0, 8, drain_body, 0)


def _sc_gather(sidx, feats):
    mesh = plsc.VectorSubcoreMesh(core_axis_name="c", subcore_axis_name="s")
    f = pl.kernel(
        _gat_body,
        out_type=jax.ShapeDtypeStruct((B, K, C), jnp.float32),
        mesh=mesh,
        compiler_params=pltpu.CompilerParams(use_tc_tiling_on_sc=False,
                                             needs_layout_passes=False),
        scratch_types=[
            pltpu.VMEM((PPT,), jnp.int32),
            pltpu.VMEM((PPT + 16,), jnp.int32),
            pltpu.VMEM((PPT + 16,), jnp.int32),
            pltpu.VMEM((PPT + 16,), jnp.int32),
            pltpu.VMEM((PPT + 16,), jnp.float32),
            pltpu.VMEM((PPT + 16,), jnp.float32),
            pltpu.VMEM((PPT + 16,), jnp.float32),
            pltpu.VMEM((PPT + 16,), jnp.float32),
            pltpu.VMEM((8, C, 2, 16), jnp.float32),
            pltpu.VMEM((8, C), jnp.float32),
            pltpu.SemaphoreType.DMA((8,)),
            pltpu.SemaphoreType.DMA((8,)),
        ],
    )
    return f(sidx, feats)


def kernel(pred_mask, features, N):
    ukeys = _stage1(pred_mask).reshape(B, HW)
    vals, ysn, xsn, sidx = _sc_select(ukeys)
    samp = _sc_gather(sidx, features)
    out = jnp.concatenate(
        [vals[..., None], ysn[..., None], xsn[..., None], samp], axis=-1)
    return out


# speedup vs baseline: 4.9101x; 1.1589x over previous
import jax
import jax.numpy as jnp
from jax import lax
from jax.experimental import pallas as pl
from jax.experimental.pallas import tpu as pltpu
from jax.experimental.pallas import tpu_sc as plsc

_NEG = -1e9
B, C, H, W = 2, 96, 384, 384
HW = H * W
K = 4096
NT = 16
CH = HW // NT
NV = CH // 16
PPT = K // NT
TRASH = K
LOOKAHEAD = 6


def _stage1_body(pm_ref, ukey_ref):
    x = pm_ref[0, 0]
    unc = -jnp.abs(x)
    binm = (x > 0.0).astype(jnp.float32)

    def shift_rows(a, d, fill):
        f = jnp.full((1, a.shape[1]), fill, a.dtype)
        if d == 1:
            return jnp.concatenate([a[1:, :], f], axis=0)
        return jnp.concatenate([f, a[:-1, :]], axis=0)

    def shift_cols(a, d, fill):
        f = jnp.full((a.shape[0], 1), fill, a.dtype)
        if d == 1:
            return jnp.concatenate([a[:, 1:], f], axis=1)
        return jnp.concatenate([f, a[:, :-1]], axis=1)

    def pool3(a, op, fill):
        h = op(a, op(shift_cols(a, 1, fill), shift_cols(a, -1, fill)))
        return op(h, op(shift_rows(h, 1, fill), shift_rows(h, -1, fill)))

    dil = pool3(binm, jnp.maximum, -jnp.inf)
    ero = pool3(binm, jnp.minimum, jnp.inf)
    edge = (dil != ero).astype(jnp.float32)
    edge2 = pool3(edge, jnp.maximum, -jnp.inf) > 0.0
    masked = jnp.where(edge2, unc, jnp.full_like(unc, _NEG))
    ukey_ref[0] = lax.bitcast_convert_type(masked, jnp.int32) & jnp.int32(0x7FFFFFFF)


def _stage1(pred_mask):
    return pl.pallas_call(
        _stage1_body,
        grid=(B,),
        in_specs=[pl.BlockSpec((1, 1, H, W), lambda b: (b, 0, 0, 0))],
        out_specs=pl.BlockSpec((1, H, W), lambda b: (b, 0, 0)),
        out_shape=jax.ShapeDtypeStruct((B, H, W), jnp.int32),
    )(pred_mask)


def _sel_body(ukeys, vals_o, ys_o, xs_o, sidx_o,
              keys_v, hist_v, lhist_v, ghist_v, sel_k_v, sel_i_v, eq_i_v,
              sidx_v, tkey_v, cnts_v, tmp16_v, sk_v, si_v, pk_v, pi_v,
              vals_v, ysn_v, xsn_v,
              hist_s, cnt_s, gk_s, gi_s, exk_s, exi_s, sem_d):
    cid = lax.axis_index("c")
    sid = lax.axis_index("s")
    lanes = lax.iota(jnp.int32, 16)
    z16 = jnp.zeros((16,), jnp.int32)
    o16 = z16 + 1
    ones_i = o16

    pltpu.sync_copy(ukeys.at[cid, pl.ds(sid * CH, CH)], keys_v)

    prefix = jnp.int32(0)
    kk = jnp.int32(K)
    for p in range(4):
        s = 24 - 8 * p

        def zero_body(i, _):
            hist_v[pl.ds(i * 16, 16)] = z16
            return 0
        lax.fori_loop(0, 256, zero_body, 0)

        def scan_body(i, _):
            v = keys_v[pl.ds(i * 16, 16)]
            dig = (v >> s) & 255
            if p == 0:
                act = v >= 0
            else:
                act = (v >> (s + 8)) == prefix
            plsc.addupdate_scatter(hist_v, [lanes * 256 + dig], ones_i, mask=act)
            return 0
        lax.fori_loop(0, NV, scan_body, 0)

        def red_body(j, _):
            acc = z16
            for l in range(16):
                acc = acc + hist_v[pl.ds(l * 256 + j * 16, 16)]
            lhist_v[pl.ds(j * 16, 16)] = acc
            return 0
        lax.fori_loop(0, 16, red_body, 0)

        pltpu.sync_copy(lhist_v, hist_s.at[sid])
        plsc.subcore_barrier()
        pltpu.sync_copy(hist_s, ghist_v)
        plsc.subcore_barrier()

        def find_body(j, carry):
            csum, dstar, below = carry
            g = z16
            for l in range(16):
                g = g + ghist_v[l, pl.ds(j * 16, 16)]
            cum = jnp.cumsum(g) + csum
            mlt = cum < kk
            dstar = dstar + jnp.sum(mlt.astype(jnp.int32))
            below = below + jnp.sum(jnp.where(mlt, g, 0))
            csum = csum + jnp.sum(g)
            return csum, dstar, below
        _, dstar, below = lax.fori_loop(
            0, 16, find_body, (jnp.int32(0), jnp.int32(0), jnp.int32(0)))
        prefix = prefix * 256 + dstar
        kk = kk - below

    T = prefix
    need_eq = kk

    def comp_body(i, carry):
        c_lt, c_eq = carry
        v = keys_v[pl.ds(i * 16, 16)]
        idxv = sid * CH + i * 16 + lanes
        m_lt = v < T
        m_eq = v == T
        plsc.store_compressed(sel_k_v.at[pl.ds(c_lt, 16)], v, mask=m_lt)
        plsc.store_compressed(sel_i_v.at[pl.ds(c_lt, 16)], idxv, mask=m_lt)
        plsc.store_compressed(eq_i_v.at[pl.ds(c_eq, 16)], idxv, mask=m_eq)
        c_lt = c_lt + jnp.sum(m_lt.astype(jnp.int32))
        c_eq = c_eq + jnp.sum(m_eq.astype(jnp.int32))
        return c_lt, c_eq
    c_lt, c_eq = lax.fori_loop(0, NV, comp_body, (jnp.int32(0), jnp.int32(0)))

    tmp16_v[pl.ds(0, 16)] = jnp.where(lanes == 0, c_lt,
                                      jnp.where(lanes == 1, c_eq, 0))
    pltpu.sync_copy(tmp16_v, cnt_s.at[sid])
    plsc.subcore_barrier()
    pltpu.sync_copy(cnt_s, cnts_v)

    nlt_vec = plsc.load_gather(cnts_v, [lanes, z16])
    neq_vec = plsc.load_gather(cnts_v, [lanes, o16])
    before = lanes < sid
    off_lt = jnp.sum(jnp.where(before, nlt_vec, 0))
    n_lt_all = jnp.sum(nlt_vec)
    off_eq = jnp.sum(jnp.where(before, neq_vec, 0))
    m_me = jnp.clip(need_eq - off_eq, 0, c_eq)

    def fill_tk(i, _):
        tkey_v[pl.ds(i * 16, 16)] = z16 + T
        return 0
    lax.fori_loop(0, 8, fill_tk, 0)

    def pack_chunks(n_items, dst_base, src_k, src_i):
        nch = (n_items + 127) // 128

        def chunk_body(cc, _):
            rem = n_items - cc * 128
            for g in range(8):
                rvec = g * 16 + lanes
                dest = jnp.where(rvec < rem, dst_base + cc * 128 + rvec,
                                 TRASH + rvec)
                sidx_v[0, pl.ds(g * 16, 16)] = dest
            if src_k is None:
                pltpu.async_copy(tkey_v, gk_s.at[sidx_v.at[0]],
                                 sem_d.at[0]).wait()
            else:
                pltpu.async_copy(src_k.at[pl.ds(cc * 128, 128)],
                                 gk_s.at[sidx_v.at[0]], sem_d.at[0]).wait()
            pltpu.async_copy(src_i.at[pl.ds(cc * 128, 128)],
                             gi_s.at[sidx_v.at[0]], sem_d.at[0]).wait()
            return 0
        lax.fori_loop(0, nch, chunk_body, 0)

    pack_chunks(c_lt, off_lt, sel_k_v, sel_i_v)
    pack_chunks(m_me, n_lt_all + off_eq, None, eq_i_v)
    plsc.subcore_barrier()

    pltpu.sync_copy(gk_s.at[pl.ds(sid * PPT, PPT)], sk_v)
    pltpu.sync_copy(gi_s.at[pl.ds(sid * PPT, PPT)], si_v)

    def keep(sel, a, b):
        return jnp.where(sel, a, b)

    for k2 in [2 << a for a in range(12)]:
        j = k2 >> 1
        while j >= 1:
            if j >= PPT:
                jb = j // PPT
                pltpu.sync_copy(sk_v, exk_s.at[sid])
                pltpu.sync_copy(si_v, exi_s.at[sid])
                plsc.subcore_barrier()
                partner = jnp.bitwise_xor(sid, jb)
                pltpu.sync_copy(exk_s.at[partner], pk_v)
                pltpu.sync_copy(exi_s.at[partner], pi_v)
                plsc.subcore_barrier()
                i_low = jnp.bitwise_and(sid, jb) == 0
                up = jnp.bitwise_and(sid * PPT, k2) == 0
                want_min = up == i_low

                def x_body(g, _):
                    a_k = sk_v[pl.ds(g * 16, 16)]
                    a_i = si_v[pl.ds(g * 16, 16)]
                    b_k = pk_v[pl.ds(g * 16, 16)]
                    b_i = pi_v[pl.ds(g * 16, 16)]
                    agtb = (a_k > b_k) | ((a_k == b_k) & (a_i > b_i))
                    sel = agtb != want_min
                    sk_v[pl.ds(g * 16, 16)] = keep(sel, a_k, b_k)
                    si_v[pl.ds(g * 16, 16)] = keep(sel, a_i, b_i)
                    return 0
                lax.fori_loop(0, 16, x_body, 0)
            elif j >= 16:
                jv = j // 16

                def m_body(pp, _):
                    v = jnp.bitwise_or(jnp.bitwise_and(pp, jv - 1),
                                       (pp & ~(jv - 1)) << 1)
                    a_k = sk_v[pl.ds(v * 16, 16)]
                    a_i = si_v[pl.ds(v * 16, 16)]
                    b_k = sk_v[pl.ds((v + jv) * 16, 16)]
                    b_i = si_v[pl.ds((v + jv) * 16, 16)]
                    agtb = (a_k > b_k) | ((a_k == b_k) & (a_i > b_i))
                    up = jnp.bitwise_and(sid * PPT + v * 16, k2) == 0
                    sel_lo = agtb != up
                    sk_v[pl.ds(v * 16, 16)] = keep(sel_lo, a_k, b_k)
                    si_v[pl.ds(v * 16, 16)] = keep(sel_lo, a_i, b_i)
                    sk_v[pl.ds((v + jv) * 16, 16)] = keep(sel_lo, b_k, a_k)
                    si_v[pl.ds((v + jv) * 16, 16)] = keep(sel_lo, b_i, a_i)
                    return 0
                lax.fori_loop(0, 8, m_body, 0)
            else:
                lxj = jnp.bitwise_xor(lanes, j)
                i_low_v = jnp.bitwise_and(lanes, j) == 0

                def v_body(g, _):
                    xk = sk_v[pl.ds(g * 16, 16)]
                    xi = si_v[pl.ds(g * 16, 16)]
                    pk = plsc.load_gather(sk_v, [g * 16 + lxj])
                    pi = plsc.load_gather(si_v, [g * 16 + lxj])
                    agtb = (xk > pk) | ((xk == pk) & (xi > pi))
                    if k2 >= 16:
                        upv = jnp.full((16,), False) | (
                            jnp.bitwise_and(sid * PPT + g * 16, k2) == 0)
                    else:
                        upv = jnp.bitwise_and(lanes, k2) == 0
                    want_min_v = upv == i_low_v
                    sel = agtb != want_min_v
                    sk_v[pl.ds(g * 16, 16)] = keep(sel, xk, pk)
                    si_v[pl.ds(g * 16, 16)] = keep(sel, xi, pi)
                    return 0
                lax.fori_loop(0, 16, v_body, 0)
            j >>= 1

    def out_body(g, _):
        ky = sk_v[pl.ds(g * 16, 16)]
        idx = si_v[pl.ds(g * 16, 16)]
        valf = plsc.bitcast(ky | jnp.int32(-2147483648), jnp.float32)
        y = ((idx >> 7) * 21846) >> 16
        x = idx - y * 384
        sl = pl.ds(g * 16, 16)
        vals_v[sl] = valf
        ysn_v[sl] = y.astype(jnp.float32) / jnp.float32(H - 1)
        xsn_v[sl] = x.astype(jnp.float32) / jnp.float32(W - 1)
        return 0
    lax.fori_loop(0, 16, out_body, 0)

    pltpu.sync_copy(vals_v, vals_o.at[cid, pl.ds(sid * PPT, PPT)])
    pltpu.sync_copy(ysn_v, ys_o.at[cid, pl.ds(sid * PPT, PPT)])
    pltpu.sync_copy(xsn_v, xs_o.at[cid, pl.ds(sid * PPT, PPT)])
    pltpu.sync_copy(si_v, sidx_o.at[cid, pl.ds(sid * PPT, PPT)])


def _sc_select(ukeys):
    mesh = plsc.VectorSubcoreMesh(core_axis_name="c", subcore_axis_name="s")
    f = pl.kernel(
        _sel_body,
        out_type=(
            jax.ShapeDtypeStruct((B, K), jnp.float32),
            jax.ShapeDtypeStruct((B, K), jnp.float32),
            jax.ShapeDtypeStruct((B, K), jnp.float32),
            jax.ShapeDtypeStruct((B, K), jnp.int32),
        ),
        mesh=mesh,
        compiler_params=pltpu.CompilerParams(use_tc_tiling_on_sc=False,
                                             needs_layout_passes=False),
        scratch_types=[
            pltpu.VMEM((CH,), jnp.int32),
            pltpu.VMEM((4096,), jnp.int32),
            pltpu.VMEM((256,), jnp.int32),
            pltpu.VMEM((16, 256), jnp.int32),
            pltpu.VMEM((4224,), jnp.int32),
            pltpu.VMEM((4224,), jnp.int32),
            pltpu.VMEM((CH + 144,), jnp.int32),
            pltpu.VMEM((2, 128), jnp.int32),
            pltpu.VMEM((128,), jnp.int32),
            pltpu.VMEM((16, 16), jnp.int32),
            pltpu.VMEM((16,), jnp.int32),
            pltpu.VMEM((PPT,), jnp.int32),
            pltpu.VMEM((PPT,), jnp.int32),
            pltpu.VMEM((PPT,), jnp.int32),
            pltpu.VMEM((PPT,), jnp.int32),
            pltpu.VMEM((PPT,), jnp.float32),
            pltpu.VMEM((PPT,), jnp.float32),
            pltpu.VMEM((PPT,), jnp.float32),
            pltpu.VMEM_SHARED((16, 256), jnp.int32),
            pltpu.VMEM_SHARED((16, 16), jnp.int32),
            pltpu.VMEM_SHARED((K + 128,), jnp.int32),
            pltpu.VMEM_SHARED((K + 128,), jnp.int32),
            pltpu.VMEM_SHARED((16, 256), jnp.int32),
            pltpu.VMEM_SHARED((16, 256), jnp.int32),
            pltpu.SemaphoreType.DMA((2,)),
        ],
    )
    return f(ukeys)


NB = 8
NBATCH = PPT // NB
ROWS = NB * 192
VROWS = (C * H * W) // 16


def _gat_body(sidx, ftab, samp_o,
              si_v, ys0_v, xs0_v, ca_v, b0_v, w00_v, w01_v, w10_v, w11_v,
              iv, gbuf2, srow2, ivf, fixbuf2, fixrow, sem_g, sem_o, sem_f):
    cid = lax.axis_index("c")
    sid = lax.axis_index("s")
    lanes = lax.iota(jnp.int32, 16)
    z16 = jnp.zeros((16,), jnp.int32)
    o16 = z16 + 1

    pltpu.sync_copy(sidx.at[cid, pl.ds(sid * PPT, PPT)], si_v)

    def parm_body(g, _):
        idx = si_v[pl.ds(g * 16, 16)]
        y = ((idx >> 7) * 21846) >> 16
        x = idx - y * 384
        ysn = y.astype(jnp.float32) / jnp.float32(H - 1)
        xsn = x.astype(jnp.float32) / jnp.float32(W - 1)
        gx = 2.0 * ysn - 1.0
        gy = 2.0 * xsn - 1.0
        iy = ((gy + 1.0) * H - 1.0) / 2.0
        ix = ((gx + 1.0) * W - 1.0) / 2.0
        y0 = (iy + 1.0).astype(jnp.int32) - 1
        x0 = (ix + 1.0).astype(jnp.int32) - 1
        wy1 = iy - y0.astype(jnp.float32)
        wx1 = ix - x0.astype(jnp.float32)
        wy0 = 1.0 - wy1
        wx0 = 1.0 - wx1
        ys0 = jnp.clip(y0, 0, H - 2)
        xs0 = jnp.clip(x0, 0, W - 2)
        zf = jnp.zeros((16,), jnp.float32)
        wyA = jnp.where(y0 == ys0, wy0, zf) + jnp.where(y0 + 1 == ys0, wy1, zf)
        wyB = (jnp.where(y0 == ys0 + 1, wy0, zf)
               + jnp.where(y0 + 1 == ys0 + 1, wy1, zf))
        wxA = jnp.where(x0 == xs0, wx0, zf) + jnp.where(x0 + 1 == xs0, wx1, zf)
        wxB = (jnp.where(x0 == xs0 + 1, wx0, zf)
               + jnp.where(x0 + 1 == xs0 + 1, wx1, zf))
        xa = xs0 & ~15
        sl = pl.ds(g * 16, 16)
        ys0_v[sl] = ys0
        xs0_v[sl] = xs0
        ca_v[sl] = xs0 - xa
        b0_v[sl] = cid * VROWS + ys0 * (W // 16) + (xa >> 4)
        w00_v[sl] = wxA * wyA
        w01_v[sl] = wxB * wyA
        w10_v[sl] = wxA * wyB
        w11_v[sl] = wxB * wyB
        return 0
    lax.fori_loop(0, 16, parm_body, 0)

    def sld(ref, i):
        return ref[pl.ds(i, 16)][0]

    def build_iv(bb, buf):
        def bld(pb, _):
            b0 = sld(b0_v, bb * NB + pb)
            b1 = b0 + (W // 16)
            base = pb * 192
            for o, bx in [(0, 0), (96, 1)]:
                for cb in range(6):
                    cv = (cb * 16 + lanes) * (HW // 16)
                    pos = base + o + cb * 16
                    iv[buf, pos >> 7, pl.ds(jnp.bitwise_and(pos, 127), 16)] = (
                        cv + (b0 if bx == 0 else b1))
            return 0
        lax.fori_loop(0, NB, bld, 0)

    def g_start(bb):
        buf = jnp.bitwise_and(bb, 1)
        for ch in range(ROWS // 128):
            pltpu.make_async_copy(
                ftab.at[iv.at[buf, ch]],
                gbuf2.at[buf, pl.ds(ch * 128, 128)],
                sem_g.at[buf]).start()

    def g_wait(bb):
        buf = jnp.bitwise_and(bb, 1)
        for ch in range(ROWS // 128):
            pltpu.make_async_copy(
                ftab.at[iv.at[buf, ch]],
                gbuf2.at[buf, pl.ds(ch * 128, 128)],
                sem_g.at[buf]).wait()

    def o_copy(bb):
        buf = jnp.bitwise_and(bb, 1)
        return pltpu.make_async_copy(
            srow2.at[buf], samp_o.at[cid, pl.ds(sid * PPT + bb * NB, NB)],
            sem_o.at[buf])

    build_iv(0, 0)
    g_start(0)

    def gb(bb, _):
        buf = jnp.bitwise_and(bb, 1)

        @pl.when(bb + 1 < NBATCH)
        def _():
            build_iv(bb + 1, 1 - buf)
            g_start(bb + 1)
        g_wait(bb)

        @pl.when(bb >= 2)
        def _():
            o_copy(bb - 2).wait()
        bufv = z16 + buf

        def cpt(pb, _):
            pgl = bb * NB + pb
            w00 = sld(w00_v, pgl)
            w01 = sld(w01_v, pgl)
            w10 = sld(w10_v, pgl)
            w11 = sld(w11_v, pgl)
            cav = z16 + sld(ca_v, pgl)
            cav1 = jnp.minimum(cav + 1, 15)
            base = pb * 192
            for cb in range(6):
                r0 = base + cb * 16 + lanes
                r1 = r0 + 96
                v00 = plsc.load_gather(gbuf2, [bufv, r0, cav])
                v01 = plsc.load_gather(gbuf2, [bufv, r0, cav1])
                v10 = plsc.load_gather(gbuf2, [bufv, r1, cav])
                v11 = plsc.load_gather(gbuf2, [bufv, r1, cav1])
                acc = v00 * w00 + v01 * w01 + v10 * w10 + v11 * w11
                srow2[buf, pb, pl.ds(cb * 16, 16)] = acc
            return 0
        lax.fori_loop(0, NB, cpt, 0)
        o_copy(bb).start()
        return 0
    lax.fori_loop(0, NBATCH, gb, 0)
    o_copy(NBATCH - 2).wait()
    o_copy(NBATCH - 1).wait()

    def fx(p, _):
        ca = sld(ca_v, p)

        @pl.when(ca == 15)
        def _():
            b0 = sld(b0_v, p)
            b1 = b0 + (W // 16)
            for cb in range(6):
                cv = (cb * 16 + lanes) * (HW // 16)
                for o, add in [(0, 0), (96, 0), (192, 1), (288, 1)]:
                    pos = o + cb * 16
                    ivf[pos >> 7, pl.ds(pos & 127, 16)] = (
                        cv + (b0 if o in (0, 192) else b1) + add)
            for ch in range(3):
                pltpu.make_async_copy(
                    ftab.at[ivf.at[ch]],
                    fixbuf2.at[pl.ds(ch * 128, 128)], sem_f).start()
            for ch in range(3):
                pltpu.make_async_copy(
                    ftab.at[ivf.at[ch]],
                    fixbuf2.at[pl.ds(ch * 128, 128)], sem_f).wait()
            w00 = sld(w00_v, p)
            w01 = sld(w01_v, p)
            w10 = sld(w10_v, p)
            w11 = sld(w11_v, p)
            c15 = z16 + 15
            for cb in range(6):
                cvec = cb * 16 + lanes
                v00 = plsc.load_gather(fixbuf2, [cvec, c15])
                v01 = plsc.load_gather(fixbuf2, [192 + cvec, z16])
                v10 = plsc.load_gather(fixbuf2, [96 + cvec, c15])
                v11 = plsc.load_gather(fixbuf2, [288 + cvec, z16])
                acc = v00 * w00 + v01 * w01 + v10 * w10 + v11 * w11
                fixrow[pl.ds(cb * 16, 16)] = acc
            pltpu.sync_copy(fixrow, samp_o.at[cid, sid * PPT + p])
        return 0
    lax.fori_loop(0, PPT, fx, 0)


def _sc_gather(sidx, ftab):
    mesh = plsc.VectorSubcoreMesh(core_axis_name="c", subcore_axis_name="s")
    f = pl.kernel(
        _gat_body,
        out_type=jax.ShapeDtypeStruct((B, K, C), jnp.float32),
        mesh=mesh,
        compiler_params=pltpu.CompilerParams(use_tc_tiling_on_sc=False,
                                             needs_layout_passes=False),
        scratch_types=[
            pltpu.VMEM((PPT,), jnp.int32),
            pltpu.VMEM((PPT + 16,), jnp.int32),
            pltpu.VMEM((PPT + 16,), jnp.int32),
            pltpu.VMEM((PPT + 16,), jnp.int32),
            pltpu.VMEM((PPT + 16,), jnp.int32),
            pltpu.VMEM((PPT + 16,), jnp.float32),
            pltpu.VMEM((PPT + 16,), jnp.float32),
            pltpu.VMEM((PPT + 16,), jnp.float32),
            pltpu.VMEM((PPT + 16,), jnp.float32),
            pltpu.VMEM((2, ROWS // 128, 128), jnp.int32),
            pltpu.VMEM((2, ROWS, 16), jnp.float32),
            pltpu.VMEM((2, NB, C), jnp.float32),
            pltpu.VMEM((3, 128), jnp.int32),
            pltpu.VMEM((384, 16), jnp.float32),
            pltpu.VMEM((C,), jnp.float32),
            pltpu.SemaphoreType.DMA((2,)),
            pltpu.SemaphoreType.DMA((2,)),
            pltpu.SemaphoreType.DMA,
        ],
    )
    return f(sidx, ftab)


def kernel(pred_mask, features, N):
    ukeys = _stage1(pred_mask).reshape(B, HW)
    vals, ysn, xsn, sidx = _sc_select(ukeys)
    ftab = features.reshape((B * C * H * W) // 16, 16)
    samp = _sc_gather(sidx, ftab)
    out = jnp.concatenate(
        [vals[..., None], ysn[..., None], xsn[..., None], samp], axis=-1)
    return out


# speedup vs baseline: 4.9550x; 1.0091x over previous
import jax
import jax.numpy as jnp
from jax import lax
from jax.experimental import pallas as pl
from jax.experimental.pallas import tpu as pltpu
from jax.experimental.pallas import tpu_sc as plsc

_NEG = -1e9
B, C, H, W = 2, 96, 384, 384
HW = H * W
K = 4096
NT = 16
CH = HW // NT
NV = CH // 16
PPT = K // NT
TRASH = K
LOOKAHEAD = 6


def _stage1_body(pm_ref, ukey_ref):
    x = pm_ref[0, 0]
    unc = -jnp.abs(x)
    binm = (x > 0.0).astype(jnp.float32)

    def shift_rows(a, d, fill):
        f = jnp.full((1, a.shape[1]), fill, a.dtype)
        if d == 1:
            return jnp.concatenate([a[1:, :], f], axis=0)
        return jnp.concatenate([f, a[:-1, :]], axis=0)

    def shift_cols(a, d, fill):
        f = jnp.full((a.shape[0], 1), fill, a.dtype)
        if d == 1:
            return jnp.concatenate([a[:, 1:], f], axis=1)
        return jnp.concatenate([f, a[:, :-1]], axis=1)

    def pool3(a, op, fill):
        h = op(a, op(shift_cols(a, 1, fill), shift_cols(a, -1, fill)))
        return op(h, op(shift_rows(h, 1, fill), shift_rows(h, -1, fill)))

    dil = pool3(binm, jnp.maximum, -jnp.inf)
    ero = pool3(binm, jnp.minimum, jnp.inf)
    edge = (dil != ero).astype(jnp.float32)
    edge2 = pool3(edge, jnp.maximum, -jnp.inf) > 0.0
    masked = jnp.where(edge2, unc, jnp.full_like(unc, _NEG))
    ukey_ref[0] = lax.bitcast_convert_type(masked, jnp.int32) & jnp.int32(0x7FFFFFFF)


def _stage1(pred_mask):
    return pl.pallas_call(
        _stage1_body,
        grid=(B,),
        in_specs=[pl.BlockSpec((1, 1, H, W), lambda b: (b, 0, 0, 0))],
        out_specs=pl.BlockSpec((1, H, W), lambda b: (b, 0, 0)),
        out_shape=jax.ShapeDtypeStruct((B, H, W), jnp.int32),
    )(pred_mask)


def _sel_body(ukeys, vals_o, ys_o, xs_o, sidx_o,
              keys_v, hist_v, lhist_v, ghist_v, sel_k_v, sel_i_v, eq_i_v,
              sidx_v, tkey_v, cnts_v, tmp16_v, sk_v, si_v, pk_v, pi_v,
              vals_v, ysn_v, xsn_v,
              hist_s, cnt_s, gk_s, gi_s, exk_s, exi_s, sem_d):
    cid = lax.axis_index("c")
    sid = lax.axis_index("s")
    lanes = lax.iota(jnp.int32, 16)
    z16 = jnp.zeros((16,), jnp.int32)
    o16 = z16 + 1
    ones_i = o16

    pltpu.sync_copy(ukeys.at[cid, pl.ds(sid * CH, CH)], keys_v)

    prefix = jnp.int32(0)
    kk = jnp.int32(K)
    for p in range(4):
        s = 24 - 8 * p

        def zero_body(i, _):
            hist_v[pl.ds(i * 16, 16)] = z16
            return 0
        lax.fori_loop(0, 256, zero_body, 0)

        def scan_body(i, _):
            v = keys_v[pl.ds(i * 16, 16)]
            dig = (v >> s) & 255
            if p == 0:
                act = v >= 0
            else:
                act = (v >> (s + 8)) == prefix
            plsc.addupdate_scatter(hist_v, [lanes * 256 + dig], ones_i, mask=act)
            return 0
        lax.fori_loop(0, NV, scan_body, 0)

        def red_body(j, _):
            acc = z16
            for l in range(16):
                acc = acc + hist_v[pl.ds(l * 256 + j * 16, 16)]
            lhist_v[pl.ds(j * 16, 16)] = acc
            return 0
        lax.fori_loop(0, 16, red_body, 0)

        pltpu.sync_copy(lhist_v, hist_s.at[sid])
        plsc.subcore_barrier()
        pltpu.sync_copy(hist_s, ghist_v)
        plsc.subcore_barrier()

        def find_body(j, carry):
            csum, dstar, below = carry
            g = z16
            for l in range(16):
                g = g + ghist_v[l, pl.ds(j * 16, 16)]
            cum = jnp.cumsum(g) + csum
            mlt = cum < kk
            dstar = dstar + jnp.sum(mlt.astype(jnp.int32))
            below = below + jnp.sum(jnp.where(mlt, g, 0))
            csum = csum + jnp.sum(g)
            return csum, dstar, below
        _, dstar, below = lax.fori_loop(
            0, 16, find_body, (jnp.int32(0), jnp.int32(0), jnp.int32(0)))
        prefix = prefix * 256 + dstar
        kk = kk - below

    T = prefix
    need_eq = kk

    def comp_body(i, carry):
        c_lt, c_eq = carry
        v = keys_v[pl.ds(i * 16, 16)]
        idxv = sid * CH + i * 16 + lanes
        m_lt = v < T
        m_eq = v == T
        plsc.store_compressed(sel_k_v.at[pl.ds(c_lt, 16)], v, mask=m_lt)
        plsc.store_compressed(sel_i_v.at[pl.ds(c_lt, 16)], idxv, mask=m_lt)
        plsc.store_compressed(eq_i_v.at[pl.ds(c_eq, 16)], idxv, mask=m_eq)
        c_lt = c_lt + jnp.sum(m_lt.astype(jnp.int32))
        c_eq = c_eq + jnp.sum(m_eq.astype(jnp.int32))
        return c_lt, c_eq
    c_lt, c_eq = lax.fori_loop(0, NV, comp_body, (jnp.int32(0), jnp.int32(0)))

    tmp16_v[pl.ds(0, 16)] = jnp.where(lanes == 0, c_lt,
                                      jnp.where(lanes == 1, c_eq, 0))
    pltpu.sync_copy(tmp16_v, cnt_s.at[sid])
    plsc.subcore_barrier()
    pltpu.sync_copy(cnt_s, cnts_v)

    nlt_vec = plsc.load_gather(cnts_v, [lanes, z16])
    neq_vec = plsc.load_gather(cnts_v, [lanes, o16])
    before = lanes < sid
    off_lt = jnp.sum(jnp.where(before, nlt_vec, 0))
    n_lt_all = jnp.sum(nlt_vec)
    off_eq = jnp.sum(jnp.where(before, neq_vec, 0))
    m_me = jnp.clip(need_eq - off_eq, 0, c_eq)

    def fill_tk(i, _):
        tkey_v[pl.ds(i * 16, 16)] = z16 + T
        return 0
    lax.fori_loop(0, 8, fill_tk, 0)

    def pack_chunks(n_items, dst_base, src_k, src_i):
        nch = (n_items + 127) // 128

        def chunk_body(cc, _):
            rem = n_items - cc * 128
            for g in range(8):
                rvec = g * 16 + lanes
                dest = jnp.where(rvec < rem, dst_base + cc * 128 + rvec,
                                 TRASH + rvec)
                sidx_v[0, pl.ds(g * 16, 16)] = dest
            if src_k is None:
                pltpu.async_copy(tkey_v, gk_s.at[sidx_v.at[0]],
                                 sem_d.at[0]).wait()
            else:
                pltpu.async_copy(src_k.at[pl.ds(cc * 128, 128)],
                                 gk_s.at[sidx_v.at[0]], sem_d.at[0]).wait()
            pltpu.async_copy(src_i.at[pl.ds(cc * 128, 128)],
                             gi_s.at[sidx_v.at[0]], sem_d.at[0]).wait()
            return 0
        lax.fori_loop(0, nch, chunk_body, 0)

    pack_chunks(c_lt, off_lt, sel_k_v, sel_i_v)
    pack_chunks(m_me, n_lt_all + off_eq, None, eq_i_v)
    plsc.subcore_barrier()

    pltpu.sync_copy(gk_s.at[pl.ds(sid * PPT, PPT)], sk_v)
    pltpu.sync_copy(gi_s.at[pl.ds(sid * PPT, PPT)], si_v)

    def keep(sel, a, b):
        return jnp.where(sel, a, b)

    for k2 in [2 << a for a in range(12)]:
        j = k2 >> 1
        while j >= 1:
            if j >= PPT:
                jb = j // PPT
                pltpu.sync_copy(sk_v, exk_s.at[sid])
                pltpu.sync_copy(si_v, exi_s.at[sid])
                plsc.subcore_barrier()
                partner = jnp.bitwise_xor(sid, jb)
                pltpu.sync_copy(exk_s.at[partner], pk_v)
                pltpu.sync_copy(exi_s.at[partner], pi_v)
                plsc.subcore_barrier()
                i_low = jnp.bitwise_and(sid, jb) == 0
                up = jnp.bitwise_and(sid * PPT, k2) == 0
                want_min = up == i_low

                def x_body(g, _):
                    a_k = sk_v[pl.ds(g * 16, 16)]
                    a_i = si_v[pl.ds(g * 16, 16)]
                    b_k = pk_v[pl.ds(g * 16, 16)]
                    b_i = pi_v[pl.ds(g * 16, 16)]
                    agtb = (a_k > b_k) | ((a_k == b_k) & (a_i > b_i))
                    sel = agtb != want_min
                    sk_v[pl.ds(g * 16, 16)] = keep(sel, a_k, b_k)
                    si_v[pl.ds(g * 16, 16)] = keep(sel, a_i, b_i)
                    return 0
                lax.fori_loop(0, 16, x_body, 0)
            elif j >= 16:
                jv = j // 16

                def m_body(pp, _):
                    v = jnp.bitwise_or(jnp.bitwise_and(pp, jv - 1),
                                       (pp & ~(jv - 1)) << 1)
                    a_k = sk_v[pl.ds(v * 16, 16)]
                    a_i = si_v[pl.ds(v * 16, 16)]
                    b_k = sk_v[pl.ds((v + jv) * 16, 16)]
                    b_i = si_v[pl.ds((v + jv) * 16, 16)]
                    agtb = (a_k > b_k) | ((a_k == b_k) & (a_i > b_i))
                    up = jnp.bitwise_and(sid * PPT + v * 16, k2) == 0
                    sel_lo = agtb != up
                    sk_v[pl.ds(v * 16, 16)] = keep(sel_lo, a_k, b_k)
                    si_v[pl.ds(v * 16, 16)] = keep(sel_lo, a_i, b_i)
                    sk_v[pl.ds((v + jv) * 16, 16)] = keep(sel_lo, b_k, a_k)
                    si_v[pl.ds((v + jv) * 16, 16)] = keep(sel_lo, b_i, a_i)
                    return 0
                lax.fori_loop(0, 8, m_body, 0)
            else:
                lxj = jnp.bitwise_xor(lanes, j)
                i_low_v = jnp.bitwise_and(lanes, j) == 0

                def v_body(g, _):
                    xk = sk_v[pl.ds(g * 16, 16)]
                    xi = si_v[pl.ds(g * 16, 16)]
                    pk = plsc.load_gather(sk_v, [g * 16 + lxj])
                    pi = plsc.load_gather(si_v, [g * 16 + lxj])
                    agtb = (xk > pk) | ((xk == pk) & (xi > pi))
                    if k2 >= 16:
                        upv = jnp.full((16,), False) | (
                            jnp.bitwise_and(sid * PPT + g * 16, k2) == 0)
                    else:
                        upv = jnp.bitwise_and(lanes, k2) == 0
                    want_min_v = upv == i_low_v
                    sel = agtb != want_min_v
                    sk_v[pl.ds(g * 16, 16)] = keep(sel, xk, pk)
                    si_v[pl.ds(g * 16, 16)] = keep(sel, xi, pi)
                    return 0
                lax.fori_loop(0, 16, v_body, 0)
            j >>= 1

    def out_body(g, _):
        ky = sk_v[pl.ds(g * 16, 16)]
        idx = si_v[pl.ds(g * 16, 16)]
        valf = plsc.bitcast(ky | jnp.int32(-2147483648), jnp.float32)
        y = ((idx >> 7) * 21846) >> 16
        x = idx - y * 384
        sl = pl.ds(g * 16, 16)
        vals_v[sl] = valf
        ysn_v[sl] = y.astype(jnp.float32) / jnp.float32(H - 1)
        xsn_v[sl] = x.astype(jnp.float32) / jnp.float32(W - 1)
        return 0
    lax.fori_loop(0, 16, out_body, 0)

    pltpu.sync_copy(vals_v, vals_o.at[cid, pl.ds(sid * PPT, PPT)])
    pltpu.sync_copy(ysn_v, ys_o.at[cid, pl.ds(sid * PPT, PPT)])
    pltpu.sync_copy(xsn_v, xs_o.at[cid, pl.ds(sid * PPT, PPT)])
    pltpu.sync_copy(si_v, sidx_o.at[cid, pl.ds(sid * PPT, PPT)])


def _sc_select(ukeys):
    mesh = plsc.VectorSubcoreMesh(core_axis_name="c", subcore_axis_name="s")
    f = pl.kernel(
        _sel_body,
        out_type=(
            jax.ShapeDtypeStruct((B, K), jnp.float32),
            jax.ShapeDtypeStruct((B, K), jnp.float32),
            jax.ShapeDtypeStruct((B, K), jnp.float32),
            jax.ShapeDtypeStruct((B, K), jnp.int32),
        ),
        mesh=mesh,
        compiler_params=pltpu.CompilerParams(use_tc_tiling_on_sc=False,
                                             needs_layout_passes=False),
        scratch_types=[
            pltpu.VMEM((CH,), jnp.int32),
            pltpu.VMEM((4096,), jnp.int32),
            pltpu.VMEM((256,), jnp.int32),
            pltpu.VMEM((16, 256), jnp.int32),
            pltpu.VMEM((4224,), jnp.int32),
            pltpu.VMEM((4224,), jnp.int32),
            pltpu.VMEM((CH + 144,), jnp.int32),
            pltpu.VMEM((2, 128), jnp.int32),
            pltpu.VMEM((128,), jnp.int32),
            pltpu.VMEM((16, 16), jnp.int32),
            pltpu.VMEM((16,), jnp.int32),
            pltpu.VMEM((PPT,), jnp.int32),
            pltpu.VMEM((PPT,), jnp.int32),
            pltpu.VMEM((PPT,), jnp.int32),
            pltpu.VMEM((PPT,), jnp.int32),
            pltpu.VMEM((PPT,), jnp.float32),
            pltpu.VMEM((PPT,), jnp.float32),
            pltpu.VMEM((PPT,), jnp.float32),
            pltpu.VMEM_SHARED((16, 256), jnp.int32),
            pltpu.VMEM_SHARED((16, 16), jnp.int32),
            pltpu.VMEM_SHARED((K + 128,), jnp.int32),
            pltpu.VMEM_SHARED((K + 128,), jnp.int32),
            pltpu.VMEM_SHARED((16, 256), jnp.int32),
            pltpu.VMEM_SHARED((16, 256), jnp.int32),
            pltpu.SemaphoreType.DMA((2,)),
        ],
    )
    return f(ukeys)


NB = 16
NBATCH = PPT // NB
ROWS = NB * 192
VROWS = (C * H * W) // 16


def _gat_body(sidx, ftab, samp_o,
              si_v, ys0_v, xs0_v, ca_v, b0_v, w00_v, w01_v, w10_v, w11_v,
              iv, gbuf2, srow2, ivf, fixbuf2, fixrow, sem_g, sem_o, sem_f):
    cid = lax.axis_index("c")
    sid = lax.axis_index("s")
    lanes = lax.iota(jnp.int32, 16)
    z16 = jnp.zeros((16,), jnp.int32)
    o16 = z16 + 1

    pltpu.sync_copy(sidx.at[cid, pl.ds(sid * PPT, PPT)], si_v)

    def parm_body(g, _):
        idx = si_v[pl.ds(g * 16, 16)]
        y = ((idx >> 7) * 21846) >> 16
        x = idx - y * 384
        ysn = y.astype(jnp.float32) / jnp.float32(H - 1)
        xsn = x.astype(jnp.float32) / jnp.float32(W - 1)
        gx = 2.0 * ysn - 1.0
        gy = 2.0 * xsn - 1.0
        iy = ((gy + 1.0) * H - 1.0) / 2.0
        ix = ((gx + 1.0) * W - 1.0) / 2.0
        y0 = (iy + 1.0).astype(jnp.int32) - 1
        x0 = (ix + 1.0).astype(jnp.int32) - 1
        wy1 = iy - y0.astype(jnp.float32)
        wx1 = ix - x0.astype(jnp.float32)
        wy0 = 1.0 - wy1
        wx0 = 1.0 - wx1
        ys0 = jnp.clip(y0, 0, H - 2)
        xs0 = jnp.clip(x0, 0, W - 2)
        zf = jnp.zeros((16,), jnp.float32)
        wyA = jnp.where(y0 == ys0, wy0, zf) + jnp.where(y0 + 1 == ys0, wy1, zf)
        wyB = (jnp.where(y0 == ys0 + 1, wy0, zf)
               + jnp.where(y0 + 1 == ys0 + 1, wy1, zf))
        wxA = jnp.where(x0 == xs0, wx0, zf) + jnp.where(x0 + 1 == xs0, wx1, zf)
        wxB = (jnp.where(x0 == xs0 + 1, wx0, zf)
               + jnp.where(x0 + 1 == xs0 + 1, wx1, zf))
        xa = xs0 & ~15
        sl = pl.ds(g * 16, 16)
        ys0_v[sl] = ys0
        xs0_v[sl] = xs0
        ca_v[sl] = xs0 - xa
        b0_v[sl] = cid * VROWS + ys0 * (W // 16) + (xa >> 4)
        w00_v[sl] = wxA * wyA
        w01_v[sl] = wxB * wyA
        w10_v[sl] = wxA * wyB
        w11_v[sl] = wxB * wyB
        return 0
    lax.fori_loop(0, 16, parm_body, 0)

    def sld(ref, i):
        return ref[pl.ds(i, 16)][0]

    def build_iv(bb, buf):
        def bld(pb, _):
            b0 = sld(b0_v, bb * NB + pb)
            b1 = b0 + (W // 16)
            base = pb * 192
            for o, bx in [(0, 0), (96, 1)]:
                for cb in range(6):
                    cv = (cb * 16 + lanes) * (HW // 16)
                    pos = base + o + cb * 16
                    iv[buf, pos >> 7, pl.ds(jnp.bitwise_and(pos, 127), 16)] = (
                        cv + (b0 if bx == 0 else b1))
            return 0
        lax.fori_loop(0, NB, bld, 0)

    def g_start(bb):
        buf = jnp.bitwise_and(bb, 1)
        for ch in range(ROWS // 128):
            pltpu.make_async_copy(
                ftab.at[iv.at[buf, ch]],
                gbuf2.at[buf, pl.ds(ch * 128, 128)],
                sem_g.at[buf]).start()

    def g_wait(bb):
        buf = jnp.bitwise_and(bb, 1)
        for ch in range(ROWS // 128):
            pltpu.make_async_copy(
                ftab.at[iv.at[buf, ch]],
                gbuf2.at[buf, pl.ds(ch * 128, 128)],
                sem_g.at[buf]).wait()

    def o_copy(bb):
        buf = jnp.bitwise_and(bb, 1)
        return pltpu.make_async_copy(
            srow2.at[buf], samp_o.at[cid, pl.ds(sid * PPT + bb * NB, NB)],
            sem_o.at[buf])

    build_iv(0, 0)
    g_start(0)

    def gb(bb, _):
        buf = jnp.bitwise_and(bb, 1)

        @pl.when(bb + 1 < NBATCH)
        def _():
            build_iv(bb + 1, 1 - buf)
            g_start(bb + 1)
        g_wait(bb)

        @pl.when(bb >= 2)
        def _():
            o_copy(bb - 2).wait()
        bufv = z16 + buf

        def cpt(pb, _):
            pgl = bb * NB + pb
            w00 = sld(w00_v, pgl)
            w01 = sld(w01_v, pgl)
            w10 = sld(w10_v, pgl)
            w11 = sld(w11_v, pgl)
            cav = z16 + sld(ca_v, pgl)
            cav1 = jnp.minimum(cav + 1, 15)
            base = pb * 192
            for cb in range(6):
                r0 = base + cb * 16 + lanes
                r1 = r0 + 96
                v00 = plsc.load_gather(gbuf2, [bufv, r0, cav])
                v01 = plsc.load_gather(gbuf2, [bufv, r0, cav1])
                v10 = plsc.load_gather(gbuf2, [bufv, r1, cav])
                v11 = plsc.load_gather(gbuf2, [bufv, r1, cav1])
                acc = v00 * w00 + v01 * w01 + v10 * w10 + v11 * w11
                srow2[buf, pb, pl.ds(cb * 16, 16)] = acc
            return 0
        lax.fori_loop(0, NB, cpt, 0)
        o_copy(bb).start()
        return 0
    lax.fori_loop(0, NBATCH, gb, 0)
    o_copy(NBATCH - 2).wait()
    o_copy(NBATCH - 1).wait()

    def fx(p, _):
        ca = sld(ca_v, p)

        @pl.when(ca == 15)
        def _():
            b0 = sld(b0_v, p)
            b1 = b0 + (W // 16)
            for cb in range(6):
                cv = (cb * 16 + lanes) * (HW // 16)
                for o, add in [(0, 0), (96, 0), (192, 1), (288, 1)]:
                    pos = o + cb * 16
                    ivf[pos >> 7, pl.ds(pos & 127, 16)] = (
                        cv + (b0 if o in (0, 192) else b1) + add)
            for ch in range(3):
                pltpu.make_async_copy(
                    ftab.at[ivf.at[ch]],
                    fixbuf2.at[pl.ds(ch * 128, 128)], sem_f).start()
            for ch in range(3):
                pltpu.make_async_copy(
                    ftab.at[ivf.at[ch]],
                    fixbuf2.at[pl.ds(ch * 128, 128)], sem_f).wait()
            w00 = sld(w00_v, p)
            w01 = sld(w01_v, p)
            w10 = sld(w10_v, p)
            w11 = sld(w11_v, p)
            c15 = z16 + 15
            for cb in range(6):
                cvec = cb * 16 + lanes
                v00 = plsc.load_gather(fixbuf2, [cvec, c15])
                v01 = plsc.load_gather(fixbuf2, [192 + cvec, z16])
                v10 = plsc.load_gather(fixbuf2, [96 + cvec, c15])
                v11 = plsc.load_gather(fixbuf2, [288 + cvec, z16])
                acc = v00 * w00 + v01 * w01 + v10 * w10 + v11 * w11
                fixrow[pl.ds(cb * 16, 16)] = acc
            pltpu.sync_copy(fixrow, samp_o.at[cid, sid * PPT + p])
        return 0
    lax.fori_loop(0, PPT, fx, 0)


def _sc_gather(sidx, ftab):
    mesh = plsc.VectorSubcoreMesh(core_axis_name="c", subcore_axis_name="s")
    f = pl.kernel(
        _gat_body,
        out_type=jax.ShapeDtypeStruct((B, K, C), jnp.float32),
        mesh=mesh,
        compiler_params=pltpu.CompilerParams(use_tc_tiling_on_sc=False,
                                             needs_layout_passes=False),
        scratch_types=[
            pltpu.VMEM((PPT,), jnp.int32),
            pltpu.VMEM((PPT + 16,), jnp.int32),
            pltpu.VMEM((PPT + 16,), jnp.int32),
            pltpu.VMEM((PPT + 16,), jnp.int32),
            pltpu.VMEM((PPT + 16,), jnp.int32),
            pltpu.VMEM((PPT + 16,), jnp.float32),
            pltpu.VMEM((PPT + 16,), jnp.float32),
            pltpu.VMEM((PPT + 16,), jnp.float32),
            pltpu.VMEM((PPT + 16,), jnp.float32),
            pltpu.VMEM((2, ROWS // 128, 128), jnp.int32),
            pltpu.VMEM((2, ROWS, 16), jnp.float32),
            pltpu.VMEM((2, NB, C), jnp.float32),
            pltpu.VMEM((3, 128), jnp.int32),
            pltpu.VMEM((384, 16), jnp.float32),
            pltpu.VMEM((C,), jnp.float32),
            pltpu.SemaphoreType.DMA((2,)),
            pltpu.SemaphoreType.DMA((2,)),
            pltpu.SemaphoreType.DMA,
        ],
    )
    return f(sidx, ftab)


def kernel(pred_mask, features, N):
    ukeys = _stage1(pred_mask).reshape(B, HW)
    vals, ysn, xsn, sidx = _sc_select(ukeys)
    ftab = features.reshape((B * C * H * W) // 16, 16)
    samp = _sc_gather(sidx, ftab)
    out = jnp.concatenate(
        [vals[..., None], ysn[..., None], xsn[..., None], samp], axis=-1)
    return out


# speedup vs baseline: 5.2033x; 1.0501x over previous
import jax
import jax.numpy as jnp
from jax import lax
from jax.experimental import pallas as pl
from jax.experimental.pallas import tpu as pltpu
from jax.experimental.pallas import tpu_sc as plsc

_NEG = -1e9
B, C, H, W = 2, 96, 384, 384
HW = H * W
K = 4096
NT = 16
CH = HW // NT
NV = CH // 16
PPT = K // NT
TRASH = K
LOOKAHEAD = 6


def _stage1_body(pm_ref, ukey_ref):
    x = pm_ref[0, 0]
    unc = -jnp.abs(x)
    binm = (x > 0.0).astype(jnp.float32)

    def shift_rows(a, d, fill):
        f = jnp.full((1, a.shape[1]), fill, a.dtype)
        if d == 1:
            return jnp.concatenate([a[1:, :], f], axis=0)
        return jnp.concatenate([f, a[:-1, :]], axis=0)

    def shift_cols(a, d, fill):
        f = jnp.full((a.shape[0], 1), fill, a.dtype)
        if d == 1:
            return jnp.concatenate([a[:, 1:], f], axis=1)
        return jnp.concatenate([f, a[:, :-1]], axis=1)

    def pool3(a, op, fill):
        h = op(a, op(shift_cols(a, 1, fill), shift_cols(a, -1, fill)))
        return op(h, op(shift_rows(h, 1, fill), shift_rows(h, -1, fill)))

    dil = pool3(binm, jnp.maximum, -jnp.inf)
    ero = pool3(binm, jnp.minimum, jnp.inf)
    edge = (dil != ero).astype(jnp.float32)
    edge2 = pool3(edge, jnp.maximum, -jnp.inf) > 0.0
    masked = jnp.where(edge2, unc, jnp.full_like(unc, _NEG))
    ukey_ref[0] = lax.bitcast_convert_type(masked, jnp.int32) & jnp.int32(0x7FFFFFFF)


def _stage1(pred_mask):
    return pl.pallas_call(
        _stage1_body,
        grid=(B,),
        in_specs=[pl.BlockSpec((1, 1, H, W), lambda b: (b, 0, 0, 0))],
        out_specs=pl.BlockSpec((1, H, W), lambda b: (b, 0, 0)),
        out_shape=jax.ShapeDtypeStruct((B, H, W), jnp.int32),
    )(pred_mask)


def _sel_body(ukeys, skey_o, sidx_o,
              keys_v, hist_v, lhist_v, ghist_v, sel_k_v, sel_i_v, eq_i_v,
              sidx_v, tkey_v, cnts_v, tmp16_v, sk_v, si_v, pk_v, pi_v,
              hist_s, cnt_s, gk_s, gi_s, exk_s, exi_s, sem_d):
    cid = lax.axis_index("c")
    sid = lax.axis_index("s")
    lanes = lax.iota(jnp.int32, 16)
    z16 = jnp.zeros((16,), jnp.int32)
    o16 = z16 + 1
    ones_i = o16

    pltpu.sync_copy(ukeys.at[cid, pl.ds(sid * CH, CH)], keys_v)

    prefix = jnp.int32(0)
    kk = jnp.int32(K)
    for p in range(4):
        s = 24 - 8 * p

        def zero_body(i, _):
            hist_v[pl.ds(i * 16, 16)] = z16
            return 0
        lax.fori_loop(0, 256, zero_body, 0)

        def scan_body(i, _):
            v = keys_v[pl.ds(i * 16, 16)]
            dig = (v >> s) & 255
            if p == 0:
                act = v >= 0
            else:
                act = (v >> (s + 8)) == prefix
            plsc.addupdate_scatter(hist_v, [lanes * 256 + dig], ones_i, mask=act)
            return 0
        lax.fori_loop(0, NV, scan_body, 0)

        def red_body(j, _):
            acc = z16
            for l in range(16):
                acc = acc + hist_v[pl.ds(l * 256 + j * 16, 16)]
            lhist_v[pl.ds(j * 16, 16)] = acc
            return 0
        lax.fori_loop(0, 16, red_body, 0)

        pltpu.sync_copy(lhist_v, hist_s.at[sid])
        plsc.subcore_barrier()
        pltpu.sync_copy(hist_s, ghist_v)
        plsc.subcore_barrier()

        def find_body(j, carry):
            csum, dstar, below = carry
            g = z16
            for l in range(16):
                g = g + ghist_v[l, pl.ds(j * 16, 16)]
            cum = jnp.cumsum(g) + csum
            mlt = cum < kk
            dstar = dstar + jnp.sum(mlt.astype(jnp.int32))
            below = below + jnp.sum(jnp.where(mlt, g, 0))
            csum = csum + jnp.sum(g)
            return csum, dstar, below
        _, dstar, below = lax.fori_loop(
            0, 16, find_body, (jnp.int32(0), jnp.int32(0), jnp.int32(0)))
        prefix = prefix * 256 + dstar
        kk = kk - below

    T = prefix
    need_eq = kk

    def comp_body(i, carry):
        c_lt, c_eq = carry
        v = keys_v[pl.ds(i * 16, 16)]
        idxv = sid * CH + i * 16 + lanes
        m_lt = v < T
        m_eq = v == T
        plsc.store_compressed(sel_k_v.at[pl.ds(c_lt, 16)], v, mask=m_lt)
        plsc.store_compressed(sel_i_v.at[pl.ds(c_lt, 16)], idxv, mask=m_lt)
        plsc.store_compressed(eq_i_v.at[pl.ds(c_eq, 16)], idxv, mask=m_eq)
        c_lt = c_lt + jnp.sum(m_lt.astype(jnp.int32))
        c_eq = c_eq + jnp.sum(m_eq.astype(jnp.int32))
        return c_lt, c_eq
    c_lt, c_eq = lax.fori_loop(0, NV, comp_body, (jnp.int32(0), jnp.int32(0)))

    tmp16_v[pl.ds(0, 16)] = jnp.where(lanes == 0, c_lt,
                                      jnp.where(lanes == 1, c_eq, 0))
    pltpu.sync_copy(tmp16_v, cnt_s.at[sid])
    plsc.subcore_barrier()
    pltpu.sync_copy(cnt_s, cnts_v)

    nlt_vec = plsc.load_gather(cnts_v, [lanes, z16])
    neq_vec = plsc.load_gather(cnts_v, [lanes, o16])
    before = lanes < sid
    off_lt = jnp.sum(jnp.where(before, nlt_vec, 0))
    n_lt_all = jnp.sum(nlt_vec)
    off_eq = jnp.sum(jnp.where(before, neq_vec, 0))
    m_me = jnp.clip(need_eq - off_eq, 0, c_eq)

    def fill_tk(i, _):
        tkey_v[pl.ds(i * 16, 16)] = z16 + T
        return 0
    lax.fori_loop(0, 8, fill_tk, 0)

    def pack_chunks(n_items, dst_base, src_k, src_i):
        nch = (n_items + 127) // 128

        def chunk_body(cc, _):
            rem = n_items - cc * 128
            for g in range(8):
                rvec = g * 16 + lanes
                dest = jnp.where(rvec < rem, dst_base + cc * 128 + rvec,
                                 TRASH + rvec)
                sidx_v[0, pl.ds(g * 16, 16)] = dest
            if src_k is None:
                pltpu.async_copy(tkey_v, gk_s.at[sidx_v.at[0]],
                                 sem_d.at[0]).wait()
            else:
                pltpu.async_copy(src_k.at[pl.ds(cc * 128, 128)],
                                 gk_s.at[sidx_v.at[0]], sem_d.at[0]).wait()
            pltpu.async_copy(src_i.at[pl.ds(cc * 128, 128)],
                             gi_s.at[sidx_v.at[0]], sem_d.at[0]).wait()
            return 0
        lax.fori_loop(0, nch, chunk_body, 0)

    pack_chunks(c_lt, off_lt, sel_k_v, sel_i_v)
    pack_chunks(m_me, n_lt_all + off_eq, None, eq_i_v)
    plsc.subcore_barrier()

    pltpu.sync_copy(gk_s.at[pl.ds(sid * PPT, PPT)], sk_v)
    pltpu.sync_copy(gi_s.at[pl.ds(sid * PPT, PPT)], si_v)

    def keep(sel, a, b):
        return jnp.where(sel, a, b)

    for k2 in [2 << a for a in range(12)]:
        j = k2 >> 1
        while j >= 1:
            if j >= PPT:
                jb = j // PPT
                pltpu.sync_copy(sk_v, exk_s.at[sid])
                pltpu.sync_copy(si_v, exi_s.at[sid])
                plsc.subcore_barrier()
                partner = jnp.bitwise_xor(sid, jb)
                pltpu.sync_copy(exk_s.at[partner], pk_v)
                pltpu.sync_copy(exi_s.at[partner], pi_v)
                plsc.subcore_barrier()
                i_low = jnp.bitwise_and(sid, jb) == 0
                up = jnp.bitwise_and(sid * PPT, k2) == 0
                want_min = up == i_low

                def x_body(g, _):
                    a_k = sk_v[pl.ds(g * 16, 16)]
                    a_i = si_v[pl.ds(g * 16, 16)]
                    b_k = pk_v[pl.ds(g * 16, 16)]
                    b_i = pi_v[pl.ds(g * 16, 16)]
                    agtb = (a_k > b_k) | ((a_k == b_k) & (a_i > b_i))
                    sel = agtb != want_min
                    sk_v[pl.ds(g * 16, 16)] = keep(sel, a_k, b_k)
                    si_v[pl.ds(g * 16, 16)] = keep(sel, a_i, b_i)
                    return 0
                lax.fori_loop(0, 16, x_body, 0)
            elif j >= 16:
                jv = j // 16

                def m_body(pp, _):
                    v = jnp.bitwise_or(jnp.bitwise_and(pp, jv - 1),
                                       (pp & ~(jv - 1)) << 1)
                    a_k = sk_v[pl.ds(v * 16, 16)]
                    a_i = si_v[pl.ds(v * 16, 16)]
                    b_k = sk_v[pl.ds((v + jv) * 16, 16)]
                    b_i = si_v[pl.ds((v + jv) * 16, 16)]
                    agtb = (a_k > b_k) | ((a_k == b_k) & (a_i > b_i))
                    up = jnp.bitwise_and(sid * PPT + v * 16, k2) == 0
                    sel_lo = agtb != up
                    sk_v[pl.ds(v * 16, 16)] = keep(sel_lo, a_k, b_k)
                    si_v[pl.ds(v * 16, 16)] = keep(sel_lo, a_i, b_i)
                    sk_v[pl.ds((v + jv) * 16, 16)] = keep(sel_lo, b_k, a_k)
                    si_v[pl.ds((v + jv) * 16, 16)] = keep(sel_lo, b_i, a_i)
                    return 0
                lax.fori_loop(0, 8, m_body, 0)
            else:
                lxj = jnp.bitwise_xor(lanes, j)
                i_low_v = jnp.bitwise_and(lanes, j) == 0

                def v_body(g, _):
                    xk = sk_v[pl.ds(g * 16, 16)]
                    xi = si_v[pl.ds(g * 16, 16)]
                    pk = plsc.load_gather(sk_v, [g * 16 + lxj])
                    pi = plsc.load_gather(si_v, [g * 16 + lxj])
                    agtb = (xk > pk) | ((xk == pk) & (xi > pi))
                    if k2 >= 16:
                        upv = jnp.full((16,), False) | (
                            jnp.bitwise_and(sid * PPT + g * 16, k2) == 0)
                    else:
                        upv = jnp.bitwise_and(lanes, k2) == 0
                    want_min_v = upv == i_low_v
                    sel = agtb != want_min_v
                    sk_v[pl.ds(g * 16, 16)] = keep(sel, xk, pk)
                    si_v[pl.ds(g * 16, 16)] = keep(sel, xi, pi)
                    return 0
                lax.fori_loop(0, 16, v_body, 0)
            j >>= 1

    pltpu.sync_copy(sk_v, skey_o.at[cid, pl.ds(sid * PPT, PPT)])
    pltpu.sync_copy(si_v, sidx_o.at[cid, pl.ds(sid * PPT, PPT)])


def _sc_select(ukeys):
    mesh = plsc.VectorSubcoreMesh(core_axis_name="c", subcore_axis_name="s")
    f = pl.kernel(
        _sel_body,
        out_type=(
            jax.ShapeDtypeStruct((B, K), jnp.int32),
            jax.ShapeDtypeStruct((B, K), jnp.int32),
        ),
        mesh=mesh,
        compiler_params=pltpu.CompilerParams(use_tc_tiling_on_sc=False,
                                             needs_layout_passes=False),
        scratch_types=[
            pltpu.VMEM((CH,), jnp.int32),
            pltpu.VMEM((4096,), jnp.int32),
            pltpu.VMEM((256,), jnp.int32),
            pltpu.VMEM((16, 256), jnp.int32),
            pltpu.VMEM((4224,), jnp.int32),
            pltpu.VMEM((4224,), jnp.int32),
            pltpu.VMEM((CH + 144,), jnp.int32),
            pltpu.VMEM((2, 128), jnp.int32),
            pltpu.VMEM((128,), jnp.int32),
            pltpu.VMEM((16, 16), jnp.int32),
            pltpu.VMEM((16,), jnp.int32),
            pltpu.VMEM((PPT,), jnp.int32),
            pltpu.VMEM((PPT,), jnp.int32),
            pltpu.VMEM((PPT,), jnp.int32),
            pltpu.VMEM((PPT,), jnp.int32),
            pltpu.VMEM_SHARED((16, 256), jnp.int32),
            pltpu.VMEM_SHARED((16, 16), jnp.int32),
            pltpu.VMEM_SHARED((K + 128,), jnp.int32),
            pltpu.VMEM_SHARED((K + 128,), jnp.int32),
            pltpu.VMEM_SHARED((16, 256), jnp.int32),
            pltpu.VMEM_SHARED((16, 256), jnp.int32),
            pltpu.SemaphoreType.DMA((2,)),
        ],
    )
    return f(ukeys)


NB = 16
NBATCH = PPT // NB
ROWS = NB * 192
VROWS = (C * H * W) // 16


def _gat_body(sidx, skeys, ftab, samp_o,
              si_v, sk256_v, valf_v, ysnf_v, xsnf_v,
              ys0_v, xs0_v, ca_v, b0_v, w00_v, w01_v, w10_v, w11_v,
              iv, gbuf2, srow2, ivf, fixbuf2, fixrow, sem_g, sem_o, sem_f):
    cid = lax.axis_index("c")
    sid = lax.axis_index("s")
    lanes = lax.iota(jnp.int32, 16)
    z16 = jnp.zeros((16,), jnp.int32)
    o16 = z16 + 1

    pltpu.sync_copy(sidx.at[cid, pl.ds(sid * PPT, PPT)], si_v)
    pltpu.sync_copy(skeys.at[cid, pl.ds(sid * PPT, PPT)], sk256_v)

    def parm_body(g, _):
        idx = si_v[pl.ds(g * 16, 16)]
        ky = sk256_v[pl.ds(g * 16, 16)]
        y = ((idx >> 7) * 21846) >> 16
        x = idx - y * 384
        ysn = y.astype(jnp.float32) / jnp.float32(H - 1)
        xsn = x.astype(jnp.float32) / jnp.float32(W - 1)
        gx = 2.0 * ysn - 1.0
        gy = 2.0 * xsn - 1.0
        iy = ((gy + 1.0) * H - 1.0) / 2.0
        ix = ((gx + 1.0) * W - 1.0) / 2.0
        y0 = (iy + 1.0).astype(jnp.int32) - 1
        x0 = (ix + 1.0).astype(jnp.int32) - 1
        wy1 = iy - y0.astype(jnp.float32)
        wx1 = ix - x0.astype(jnp.float32)
        wy0 = 1.0 - wy1
        wx0 = 1.0 - wx1
        ys0 = jnp.clip(y0, 0, H - 2)
        xs0 = jnp.clip(x0, 0, W - 2)
        zf = jnp.zeros((16,), jnp.float32)
        wyA = jnp.where(y0 == ys0, wy0, zf) + jnp.where(y0 + 1 == ys0, wy1, zf)
        wyB = (jnp.where(y0 == ys0 + 1, wy0, zf)
               + jnp.where(y0 + 1 == ys0 + 1, wy1, zf))
        wxA = jnp.where(x0 == xs0, wx0, zf) + jnp.where(x0 + 1 == xs0, wx1, zf)
        wxB = (jnp.where(x0 == xs0 + 1, wx0, zf)
               + jnp.where(x0 + 1 == xs0 + 1, wx1, zf))
        xa = xs0 & ~15
        sl = pl.ds(g * 16, 16)
        valf_v[sl] = plsc.bitcast(ky | jnp.int32(-2147483648), jnp.float32)
        ysnf_v[sl] = ysn
        xsnf_v[sl] = xsn
        ys0_v[sl] = ys0
        xs0_v[sl] = xs0
        ca_v[sl] = xs0 - xa
        b0_v[sl] = cid * VROWS + ys0 * (W // 16) + (xa >> 4)
        w00_v[sl] = wxA * wyA
        w01_v[sl] = wxB * wyA
        w10_v[sl] = wxA * wyB
        w11_v[sl] = wxB * wyB
        return 0
    lax.fori_loop(0, 16, parm_body, 0)

    def sld(ref, i):
        return ref[pl.ds(i, 16)][0]

    def build_iv(bb, buf):
        def bld(pb, _):
            b0 = sld(b0_v, bb * NB + pb)
            b1 = b0 + (W // 16)
            base = pb * 192
            for o, bx in [(0, 0), (96, 1)]:
                for cb in range(6):
                    cv = (cb * 16 + lanes) * (HW // 16)
                    pos = base + o + cb * 16
                    iv[buf, pos >> 7, pl.ds(jnp.bitwise_and(pos, 127), 16)] = (
                        cv + (b0 if bx == 0 else b1))
            return 0
        lax.fori_loop(0, NB, bld, 0)

    def g_start(bb):
        buf = jnp.bitwise_and(bb, 1)
        for ch in range(ROWS // 128):
            pltpu.make_async_copy(
                ftab.at[iv.at[buf, ch]],
                gbuf2.at[buf, pl.ds(ch * 128, 128)],
                sem_g.at[buf]).start()

    def g_wait(bb):
        buf = jnp.bitwise_and(bb, 1)
        for ch in range(ROWS // 128):
            pltpu.make_async_copy(
                ftab.at[iv.at[buf, ch]],
                gbuf2.at[buf, pl.ds(ch * 128, 128)],
                sem_g.at[buf]).wait()

    def o_copy(bb):
        buf = jnp.bitwise_and(bb, 1)
        return pltpu.make_async_copy(
            srow2.at[buf], samp_o.at[cid, pl.ds(sid * PPT + bb * NB, NB)],
            sem_o.at[buf])

    build_iv(0, 0)
    g_start(0)

    def gb(bb, _):
        buf = jnp.bitwise_and(bb, 1)

        @pl.when(bb + 1 < NBATCH)
        def _():
            build_iv(bb + 1, 1 - buf)
            g_start(bb + 1)
        g_wait(bb)

        @pl.when(bb >= 2)
        def _():
            o_copy(bb - 2).wait()
        bufv = z16 + buf

        def cpt(pb, _):
            pgl = bb * NB + pb
            w00 = sld(w00_v, pgl)
            w01 = sld(w01_v, pgl)
            w10 = sld(w10_v, pgl)
            w11 = sld(w11_v, pgl)
            cav = z16 + sld(ca_v, pgl)
            cav1 = jnp.minimum(cav + 1, 15)
            hdr = jnp.where(lanes == 0, sld(valf_v, pgl),
                            jnp.where(lanes == 1, sld(ysnf_v, pgl),
                                      jnp.where(lanes == 2, sld(xsnf_v, pgl),
                                                0.0)))
            srow2[buf, pb, pl.ds(0, 16)] = hdr
            base = pb * 192
            for cb in range(6):
                r0 = base + cb * 16 + lanes
                r1 = r0 + 96
                v00 = plsc.load_gather(gbuf2, [bufv, r0, cav])
                v01 = plsc.load_gather(gbuf2, [bufv, r0, cav1])
                v10 = plsc.load_gather(gbuf2, [bufv, r1, cav])
                v11 = plsc.load_gather(gbuf2, [bufv, r1, cav1])
                acc = v00 * w00 + v01 * w01 + v10 * w10 + v11 * w11
                srow2[buf, pb, pl.ds(3 + cb * 16, 16)] = acc
            return 0
        lax.fori_loop(0, NB, cpt, 0)
        o_copy(bb).start()
        return 0
    lax.fori_loop(0, NBATCH, gb, 0)
    o_copy(NBATCH - 2).wait()
    o_copy(NBATCH - 1).wait()

    def fx(p, _):
        ca = sld(ca_v, p)

        @pl.when(ca == 15)
        def _():
            b0 = sld(b0_v, p)
            b1 = b0 + (W // 16)
            for cb in range(6):
                cv = (cb * 16 + lanes) * (HW // 16)
                for o, add in [(0, 0), (96, 0), (192, 1), (288, 1)]:
                    pos = o + cb * 16
                    ivf[pos >> 7, pl.ds(pos & 127, 16)] = (
                        cv + (b0 if o in (0, 192) else b1) + add)
            for ch in range(3):
                pltpu.make_async_copy(
                    ftab.at[ivf.at[ch]],
                    fixbuf2.at[pl.ds(ch * 128, 128)], sem_f).start()
            for ch in range(3):
                pltpu.make_async_copy(
                    ftab.at[ivf.at[ch]],
                    fixbuf2.at[pl.ds(ch * 128, 128)], sem_f).wait()
            w00 = sld(w00_v, p)
            w01 = sld(w01_v, p)
            w10 = sld(w10_v, p)
            w11 = sld(w11_v, p)
            hdr = jnp.where(lanes == 0, sld(valf_v, p),
                            jnp.where(lanes == 1, sld(ysnf_v, p),
                                      jnp.where(lanes == 2, sld(xsnf_v, p),
                                                0.0)))
            fixrow[pl.ds(0, 16)] = hdr
            c15 = z16 + 15
            for cb in range(6):
                cvec = cb * 16 + lanes
                v00 = plsc.load_gather(fixbuf2, [cvec, c15])
                v01 = plsc.load_gather(fixbuf2, [192 + cvec, z16])
                v10 = plsc.load_gather(fixbuf2, [96 + cvec, c15])
                v11 = plsc.load_gather(fixbuf2, [288 + cvec, z16])
                acc = v00 * w00 + v01 * w01 + v10 * w10 + v11 * w11
                fixrow[pl.ds(3 + cb * 16, 16)] = acc
            pltpu.sync_copy(fixrow, samp_o.at[cid, sid * PPT + p])
        return 0
    lax.fori_loop(0, PPT, fx, 0)


def _sc_gather(sidx, skeys, ftab):
    mesh = plsc.VectorSubcoreMesh(core_axis_name="c", subcore_axis_name="s")
    f = pl.kernel(
        _gat_body,
        out_type=jax.ShapeDtypeStruct((B, K, C + 3), jnp.float32),
        mesh=mesh,
        compiler_params=pltpu.CompilerParams(use_tc_tiling_on_sc=False,
                                             needs_layout_passes=False),
        scratch_types=[
            pltpu.VMEM((PPT,), jnp.int32),
            pltpu.VMEM((PPT,), jnp.int32),
            pltpu.VMEM((PPT + 16,), jnp.float32),
            pltpu.VMEM((PPT + 16,), jnp.float32),
            pltpu.VMEM((PPT + 16,), jnp.float32),
            pltpu.VMEM((PPT + 16,), jnp.int32),
            pltpu.VMEM((PPT + 16,), jnp.int32),
            pltpu.VMEM((PPT + 16,), jnp.int32),
            pltpu.VMEM((PPT + 16,), jnp.int32),
            pltpu.VMEM((PPT + 16,), jnp.float32),
            pltpu.VMEM((PPT + 16,), jnp.float32),
            pltpu.VMEM((PPT + 16,), jnp.float32),
            pltpu.VMEM((PPT + 16,), jnp.float32),
            pltpu.VMEM((2, ROWS // 128, 128), jnp.int32),
            pltpu.VMEM((2, ROWS, 16), jnp.float32),
            pltpu.VMEM((2, NB, C + 3), jnp.float32),
            pltpu.VMEM((3, 128), jnp.int32),
            pltpu.VMEM((384, 16), jnp.float32),
            pltpu.VMEM((C + 3,), jnp.float32),
            pltpu.SemaphoreType.DMA((2,)),
            pltpu.SemaphoreType.DMA((2,)),
            pltpu.SemaphoreType.DMA,
        ],
    )
    return f(sidx, skeys, ftab)


def kernel(pred_mask, features, N):
    ukeys = _stage1(pred_mask).reshape(B, HW)
    skeys, sidx = _sc_select(ukeys)
    ftab = features.reshape((B * C * H * W) // 16, 16)
    return _sc_gather(sidx, skeys, ftab)
